# baseline (speedup 1.0000x reference)
import functools

import jax
import jax.numpy as jnp
import numpy as np
from jax.experimental import pallas as pl
from jax.experimental.pallas import tpu as pltpu

N = 10000
E = 320000
H = 128
F = 128
NG = 50
NI = 6
G = 128
CUT = 10.0
ZMAX = 100

_LOG2 = float(np.log(2.0))


def _ssp(x):
    return jax.nn.softplus(x) - _LOG2


def _film_head_kernel(h_ref, gam_ref, bet_ref, w1_ref, b1_ref, w2_ref, b2_ref,
                      o_ref):
    h = gam_ref[...] * h_ref[...] + bet_ref[...]
    t = _ssp(jnp.dot(h, w1_ref[...], preferred_element_type=jnp.float32)
             + b1_ref[...])
    o_ref[...] = (jnp.dot(t, w2_ref[...], preferred_element_type=jnp.float32)
                  + b2_ref[...])


def _film_head(h, gam_n, bet_n, out1_w, out1_b, out2_w, out2_b):
    NP = 10240
    blk = 512
    pad = NP - h.shape[0]
    hp = jnp.pad(h, ((0, pad), (0, 0)))
    gp = jnp.pad(gam_n, ((0, pad), (0, 0)))
    bp = jnp.pad(bet_n, ((0, pad), (0, 0)))
    out = pl.pallas_call(
        _film_head_kernel,
        grid=(NP // blk,),
        in_specs=[
            pl.BlockSpec((blk, H), lambda i: (i, 0)),
            pl.BlockSpec((blk, H), lambda i: (i, 0)),
            pl.BlockSpec((blk, H), lambda i: (i, 0)),
            pl.BlockSpec((H, H // 2), lambda i: (0, 0)),
            pl.BlockSpec((H // 2,), lambda i: (0,)),
            pl.BlockSpec((H // 2, 8), lambda i: (0, 0)),
            pl.BlockSpec((8,), lambda i: (0,)),
        ],
        out_specs=pl.BlockSpec((blk, 8), lambda i: (i, 0)),
        out_shape=jax.ShapeDtypeStruct((NP, 8), jnp.float32),
    )(hp, gp, bp, out1_w, out1_b,
      jnp.pad(out2_w, ((0, 0), (0, 7))), jnp.pad(out2_b, (0, 7)))
    return out[:h.shape[0], :1]


def kernel(pos, atomic_numbers, batch, edge_index, domain_ids, emb, mlp_w1,
           mlp_b1, mlp_w2, mlp_b2, lin1_w, lin2_w, lin2_b, lin3_w, lin3_b,
           out1_w, out1_b, out2_w, out2_b, dom_emb, fp1_w, fp1_b, fp2_w,
           fp2_b, gam_w, gam_b, bet_w, bet_b):
    row = edge_index[0]
    col = edge_index[1]
    diff = pos[row] - pos[col]
    d = jnp.sqrt(jnp.sum(diff * diff, axis=-1) + 1e-12)
    offsets = jnp.linspace(0.0, CUT, NG)
    coeff = -0.5 / (offsets[1] - offsets[0]) ** 2
    rbf = jnp.exp(coeff * (d[:, None] - offsets[None, :]) ** 2)
    C = 0.5 * (jnp.cos(d * np.pi / CUT) + 1.0) * (d < CUT).astype(jnp.float32)
    h = emb[atomic_numbers]
    for i in range(NI):
        W = _ssp(rbf @ mlp_w1[i] + mlp_b1[i]) @ mlp_w2[i] + mlp_b2[i]
        W = W * C[:, None]
        xj = (h @ lin1_w[i])[col]
        msg = xj * W
        agg = jnp.zeros((N, F), jnp.float32).at[row].add(msg)
        x = _ssp(agg @ lin2_w[i] + lin2_b[i])
        x = x @ lin3_w[i] + lin3_b[i]
        h = h + x
    de = dom_emb[domain_ids]
    fc = jax.nn.relu(de @ fp1_w + fp1_b) @ fp2_w + fp2_b
    gamma = fc @ gam_w + gam_b
    beta = fc @ bet_w + bet_b
    o = _film_head(h, gamma[batch], beta[batch], out1_w, out1_b, out2_w,
                   out2_b)
    energies = jax.ops.segment_sum(o, batch, num_segments=G)
    return energies


# speedup vs baseline: 1.8764x; 1.8764x over previous
import functools

import jax
import jax.numpy as jnp
import numpy as np
from jax import lax
from jax.experimental import pallas as pl
from jax.experimental.pallas import tpu as pltpu
from jax.experimental.pallas import tpu_sc as plsc

N = 10000
E = 320000
H = 128
F = 128
NG = 50
NI = 6
G = 128
CUT = 10.0
ZMAX = 100

NC = 2
NS = 16
NW = NC * NS
CHUNK = 64
NCHUNK = 158
EPW = NCHUNK * CHUNK
EP = NW * EPW
NP = 10240
NB = 512
ROWS_PER_SUB = NP // NS

_LOG2 = float(np.log(2.0))
_STEP = CUT / (NG - 1)
_COEFF = -0.5 / _STEP ** 2


def _ssp(x):
    return jax.nn.softplus(x) - _LOG2


def _sc_dist_body(pos_hbm, row_hbm, col_hbm, d2_hbm, pos_t, rowb, colb, d2b):
    wid = lax.axis_index("s") * NC + lax.axis_index("c")
    base = wid * EPW
    pltpu.sync_copy(pos_hbm, pos_t)
    pltpu.sync_copy(row_hbm.at[pl.ds(base, EPW)], rowb)
    pltpu.sync_copy(col_hbm.at[pl.ds(base, EPW)], colb)

    def body(g, carry):
        sl = pl.ds(g * 16, 16)
        r3 = rowb[sl] * 3
        c3 = colb[sl] * 3
        dx = plsc.load_gather(pos_t, [r3]) - plsc.load_gather(pos_t, [c3])
        dy = plsc.load_gather(pos_t, [r3 + 1]) - plsc.load_gather(pos_t, [c3 + 1])
        dz = plsc.load_gather(pos_t, [r3 + 2]) - plsc.load_gather(pos_t, [c3 + 2])
        d2b[sl] = dx * dx + dy * dy + dz * dz
        return carry

    lax.fori_loop(0, EPW // 16, body, 0, unroll=2)
    pltpu.sync_copy(d2b, d2_hbm.at[pl.ds(base, EPW)])


@jax.jit
def _sc_dist(pos_flat, row_p, col_p):
    mesh = plsc.VectorSubcoreMesh(core_axis_name="c", subcore_axis_name="s",
                                  num_cores=NC, num_subcores=NS)
    return pl.kernel(
        _sc_dist_body,
        out_type=jax.ShapeDtypeStruct((EP,), jnp.float32),
        mesh=mesh,
        compiler_params=pltpu.CompilerParams(needs_layout_passes=False),
        scratch_types=[
            pltpu.VMEM((N * 3,), jnp.float32),
            pltpu.VMEM((EPW,), jnp.int32),
            pltpu.VMEM((EPW,), jnp.int32),
            pltpu.VMEM((EPW,), jnp.float32),
        ],
    )(pos_flat, row_p, col_p)


def _sc_gs_body(x1_hbm, wc_hbm, row_hbm, col_hbm, agg_hbm,
                colb, rowb, xjb, wcb, shared, gsem):
    c = lax.axis_index("c")
    s = lax.axis_index("s")
    wid = s * NC + c
    ebase = wid * EPW

    def zbody(e, carry):
        for j in range(8):
            xjb[e, pl.ds(j * 16, 16)] = jnp.zeros((16,), jnp.float32)
        return carry
    lax.fori_loop(0, CHUNK, zbody, 0, unroll=2)
    r0 = s * ROWS_PER_SUB
    for k in range(10):
        pltpu.sync_copy(xjb, shared.at[pl.ds(r0 + k * CHUNK, CHUNK)])
    plsc.subcore_barrier()

    def chunk(ch, carry):
        pltpu.sync_copy(col_hbm.at[wid, ch], colb)
        cp = pltpu.async_copy(x1_hbm.at[colb], xjb, gsem)
        pltpu.sync_copy(row_hbm.at[wid, ch], rowb)
        pltpu.sync_copy(wc_hbm.at[pl.ds(ebase + ch * CHUNK, CHUNK)], wcb)
        cp.wait()

        def mul(e, carry2):
            for j in range(8):
                sl = pl.ds(j * 16, 16)
                xjb[e, sl] = xjb[e, sl] * wcb[e, sl]
            return carry2
        lax.fori_loop(0, CHUNK, mul, 0, unroll=2)
        pltpu.sync_copy(xjb, shared.at[rowb], add=True)
        return carry

    lax.fori_loop(0, NCHUNK, chunk, 0)
    plsc.subcore_barrier()
    pltpu.sync_copy(shared.at[pl.ds(r0, ROWS_PER_SUB)],
                    agg_hbm.at[c, pl.ds(r0, ROWS_PER_SUB)])


@jax.jit
def _sc_gather_scatter(x1, wc, row3d, col3d):
    mesh = plsc.VectorSubcoreMesh(core_axis_name="c", subcore_axis_name="s",
                                  num_cores=NC, num_subcores=NS)
    return pl.kernel(
        _sc_gs_body,
        out_type=jax.ShapeDtypeStruct((NC, NP, F), jnp.float32),
        mesh=mesh,
        compiler_params=pltpu.CompilerParams(needs_layout_passes=False),
        scratch_types=[
            pltpu.VMEM((CHUNK,), jnp.int32),
            pltpu.VMEM((CHUNK,), jnp.int32),
            pltpu.VMEM((CHUNK, F), jnp.float32),
            pltpu.VMEM((CHUNK, F), jnp.float32),
            pltpu.VMEM_SHARED((NP, F), jnp.float32),
            pltpu.SemaphoreType.DMA,
        ],
    )(x1, wc, row3d, col3d)


def _tc_filter_kernel(d2_ref, w1_ref, b1_ref, w2_ref, b2_ref, wc_ref):
    i = pl.program_id(0)
    d2 = d2_ref[...]
    d = jnp.sqrt(d2 + 1e-12)
    j = lax.broadcasted_iota(jnp.int32, (NB, 64), 1).astype(jnp.float32)
    rbf = jnp.exp(_COEFF * (d - j * _STEP) ** 2)
    t = _ssp(jnp.dot(rbf, w1_ref[...], preferred_element_type=jnp.float32)
             + b1_ref[...])
    W = jnp.dot(t, w2_ref[...], preferred_element_type=jnp.float32) + b2_ref[...]
    C = 0.5 * (jnp.cos(d * (np.pi / CUT)) + 1.0)
    C = jnp.where(d < CUT, C, 0.0)
    eidx = i * NB + lax.broadcasted_iota(jnp.int32, (NB, 1), 0)
    C = jnp.where(eidx < E, C, 0.0)
    wc_ref[...] = W * C


@jax.jit
def _tc_filter(d2c, w1p, b1, w2, b2):
    return pl.pallas_call(
        _tc_filter_kernel,
        grid=(EP // NB,),
        in_specs=[
            pl.BlockSpec((NB, 1), lambda i: (i, 0)),
            pl.BlockSpec((64, F), lambda i: (0, 0)),
            pl.BlockSpec((1, F), lambda i: (0, 0)),
            pl.BlockSpec((F, F), lambda i: (0, 0)),
            pl.BlockSpec((1, F), lambda i: (0, 0)),
        ],
        out_specs=pl.BlockSpec((NB, F), lambda i: (i, 0)),
        out_shape=jax.ShapeDtypeStruct((EP, F), jnp.float32),
    )(d2c, w1p, b1, w2, b2)


def _tc_prologue_kernel(az_ref, emb_ref, l1_ref, h_ref, x1_ref):
    az = az_ref[...]
    j = lax.broadcasted_iota(jnp.int32, (NB, 128), 1).astype(jnp.float32)
    oh = (az == j).astype(jnp.float32)
    h = jnp.dot(oh, emb_ref[...], preferred_element_type=jnp.float32)
    h_ref[...] = h
    x1_ref[...] = jnp.dot(h, l1_ref[...], preferred_element_type=jnp.float32)


@jax.jit
def _tc_prologue(azf, embp, l1):
    return pl.pallas_call(
        _tc_prologue_kernel,
        grid=(NP // NB,),
        in_specs=[
            pl.BlockSpec((NB, 1), lambda i: (i, 0)),
            pl.BlockSpec((128, H), lambda i: (0, 0)),
            pl.BlockSpec((H, F), lambda i: (0, 0)),
        ],
        out_specs=[pl.BlockSpec((NB, H), lambda i: (i, 0)),
                   pl.BlockSpec((NB, F), lambda i: (i, 0))],
        out_shape=[jax.ShapeDtypeStruct((NP, H), jnp.float32),
                   jax.ShapeDtypeStruct((NP, F), jnp.float32)],
    )(azf, embp, l1)


def _tc_node_kernel(a_ref, h_ref, w2_ref, b2_ref, w3_ref, b3_ref, l1_ref,
                    hn_ref, x1_ref):
    i = pl.program_id(0)
    agg = a_ref[0] + a_ref[1]
    x = _ssp(jnp.dot(agg, w2_ref[...], preferred_element_type=jnp.float32)
             + b2_ref[...])
    x = jnp.dot(x, w3_ref[...], preferred_element_type=jnp.float32) + b3_ref[...]
    ridx = i * NB + lax.broadcasted_iota(jnp.int32, (NB, 1), 0)
    x = jnp.where(ridx < N, x, 0.0)
    hn = h_ref[...] + x
    hn_ref[...] = hn
    x1_ref[...] = jnp.dot(hn, l1_ref[...], preferred_element_type=jnp.float32)


@jax.jit
def _tc_node_update(aggp, h, w2, b2, w3, b3, l1n):
    return pl.pallas_call(
        _tc_node_kernel,
        grid=(NP // NB,),
        in_specs=[
            pl.BlockSpec((2, NB, F), lambda i: (0, i, 0)),
            pl.BlockSpec((NB, H), lambda i: (i, 0)),
            pl.BlockSpec((F, H), lambda i: (0, 0)),
            pl.BlockSpec((1, H), lambda i: (0, 0)),
            pl.BlockSpec((H, H), lambda i: (0, 0)),
            pl.BlockSpec((1, H), lambda i: (0, 0)),
            pl.BlockSpec((H, F), lambda i: (0, 0)),
        ],
        out_specs=[pl.BlockSpec((NB, H), lambda i: (i, 0)),
                   pl.BlockSpec((NB, F), lambda i: (i, 0))],
        out_shape=[jax.ShapeDtypeStruct((NP, H), jnp.float32),
                   jax.ShapeDtypeStruct((NP, F), jnp.float32)],
    )(aggp, h, w2, b2, w3, b3, l1n)


def _tc_film_kernel(did_ref, de_ref, w1_ref, b1_ref, w2_ref, b2_ref,
                    gw_ref, gb_ref, bw_ref, bb_ref, gam_ref, bet_ref):
    did = did_ref[...]
    j = lax.broadcasted_iota(jnp.int32, (G, 8), 1).astype(jnp.float32)
    oh = (did == j).astype(jnp.float32)
    de = jnp.dot(oh, de_ref[...], preferred_element_type=jnp.float32)
    fc = jax.nn.relu(jnp.dot(de, w1_ref[...], preferred_element_type=jnp.float32)
                     + b1_ref[...])
    fc = jnp.dot(fc, w2_ref[...], preferred_element_type=jnp.float32) + b2_ref[...]
    gam_ref[...] = jnp.dot(fc, gw_ref[...],
                           preferred_element_type=jnp.float32) + gb_ref[...]
    bet_ref[...] = jnp.dot(fc, bw_ref[...],
                           preferred_element_type=jnp.float32) + bb_ref[...]


@jax.jit
def _tc_film(didf, dep, fp1_w, fp1_b, fp2_w, fp2_b, gam_w, gam_b, bet_w, bet_b):
    full = lambda *s: pl.BlockSpec(s, lambda: tuple(0 for _ in s))
    return pl.pallas_call(
        _tc_film_kernel,
        in_specs=[full(G, 1), full(8, 64), full(64, 128), full(1, 128),
                  full(128, 128), full(1, 128), full(128, H), full(1, H),
                  full(128, H), full(1, H)],
        out_specs=[full(G, H), full(G, H)],
        out_shape=[jax.ShapeDtypeStruct((G, H), jnp.float32),
                   jax.ShapeDtypeStruct((G, H), jnp.float32)],
    )(didf, dep, fp1_w, fp1_b, fp2_w, fp2_b, gam_w, gam_b, bet_w, bet_b)


def _tc_final_kernel(h_ref, b_ref, gam_ref, bet_ref, w1_ref, b1_ref,
                     w2_ref, b2_ref, out_ref):
    i = pl.program_id(0)
    bf = b_ref[...]
    j = lax.broadcasted_iota(jnp.int32, (NB, G), 1).astype(jnp.float32)
    oh = (bf == j).astype(jnp.float32)
    gam_n = jnp.dot(oh, gam_ref[...], preferred_element_type=jnp.float32)
    bet_n = jnp.dot(oh, bet_ref[...], preferred_element_type=jnp.float32)
    hp = gam_n * h_ref[...] + bet_n
    t = _ssp(jnp.dot(hp, w1_ref[...], preferred_element_type=jnp.float32)
             + b1_ref[...])
    o = jnp.dot(t, w2_ref[...], preferred_element_type=jnp.float32) + b2_ref[...]
    part = lax.dot_general(oh, o, (((0,), (0,)), ((), ())),
                           preferred_element_type=jnp.float32)

    @pl.when(i == 0)
    def _():
        out_ref[...] = jnp.zeros_like(out_ref)

    out_ref[...] += part


@jax.jit
def _tc_final(h, batchf, gamma, beta, out1_w, out1_b, out2_wp, out2_bp):
    return pl.pallas_call(
        _tc_final_kernel,
        grid=(NP // NB,),
        in_specs=[
            pl.BlockSpec((NB, H), lambda i: (i, 0)),
            pl.BlockSpec((NB, 1), lambda i: (i, 0)),
            pl.BlockSpec((G, H), lambda i: (0, 0)),
            pl.BlockSpec((G, H), lambda i: (0, 0)),
            pl.BlockSpec((H, H // 2), lambda i: (0, 0)),
            pl.BlockSpec((1, H // 2), lambda i: (0, 0)),
            pl.BlockSpec((H // 2, 8), lambda i: (0, 0)),
            pl.BlockSpec((1, 8), lambda i: (0, 0)),
        ],
        out_specs=pl.BlockSpec((G, 8), lambda i: (0, 0)),
        out_shape=jax.ShapeDtypeStruct((G, 8), jnp.float32),
    )(h, batchf, gamma, beta, out1_w, out1_b, out2_wp, out2_bp)


def kernel(pos, atomic_numbers, batch, edge_index, domain_ids, emb, mlp_w1,
           mlp_b1, mlp_w2, mlp_b2, lin1_w, lin2_w, lin2_b, lin3_w, lin3_b,
           out1_w, out1_b, out2_w, out2_b, dom_emb, fp1_w, fp1_b, fp2_w,
           fp2_b, gam_w, gam_b, bet_w, bet_b):
    npad = EP - E
    pad_idx = (jnp.arange(npad, dtype=jnp.int32) * 7) % N
    row_p = jnp.concatenate([edge_index[0].astype(jnp.int32), pad_idx])
    col_p = jnp.concatenate([edge_index[1].astype(jnp.int32), pad_idx])
    row3d = row_p.reshape(NW, NCHUNK, CHUNK)
    col3d = col_p.reshape(NW, NCHUNK, CHUNK)
    pos_flat = pos.reshape(-1)

    azf = jnp.pad(atomic_numbers.astype(jnp.float32), (0, NP - N),
                  constant_values=float(ZMAX)).reshape(NP, 1)
    batchf = jnp.pad(batch.astype(jnp.float32), (0, NP - N),
                     constant_values=float(G)).reshape(NP, 1)
    didf = domain_ids.astype(jnp.float32).reshape(G, 1)

    w1p = jnp.pad(mlp_w1, ((0, 0), (0, 64 - NG), (0, 0)))
    embp = jnp.pad(emb, ((0, 128 - ZMAX), (0, 0)))
    dep = jnp.pad(dom_emb, ((0, 3), (0, 0)))
    out2_wp = jnp.pad(out2_w, ((0, 0), (0, 7)))
    out2_bp = jnp.pad(out2_b, (0, 7)).reshape(1, 8)
    r2 = lambda v: v.reshape(1, -1)

    d2 = _sc_dist(pos_flat, row_p, col_p)
    d2c = d2.reshape(EP, 1)
    h, x1 = _tc_prologue(azf, embp, lin1_w[0])
    for i in range(NI):
        wc = _tc_filter(d2c, w1p[i], r2(mlp_b1[i]), mlp_w2[i], r2(mlp_b2[i]))
        aggp = _sc_gather_scatter(x1, wc, row3d, col3d)
        h, x1 = _tc_node_update(aggp, h, lin2_w[i], r2(lin2_b[i]),
                                lin3_w[i], r2(lin3_b[i]), lin1_w[(i + 1) % NI])
    gamma, beta = _tc_film(didf, dep, fp1_w, r2(fp1_b), fp2_w, r2(fp2_b),
                           gam_w, r2(gam_b), bet_w, r2(bet_b))
    eng = _tc_final(h, batchf, gamma, beta, out1_w, r2(out1_b),
                    out2_wp, out2_bp)
    return eng[:, :1]


# speedup vs baseline: 1.9632x; 1.0462x over previous
import functools

import jax
import jax.numpy as jnp
import numpy as np
from jax import lax
from jax.experimental import pallas as pl
from jax.experimental.pallas import tpu as pltpu
from jax.experimental.pallas import tpu_sc as plsc

N = 10000
E = 320000
H = 128
F = 128
NG = 50
NI = 6
G = 128
CUT = 10.0
ZMAX = 100

NC = 2
NS = 16
NW = NC * NS
CHUNK = 64
NCHUNK = 160
IBLK = 32
EPW = NCHUNK * CHUNK
EP = NW * EPW
NP = 10240
NB = 512
ROWS_PER_SUB = NP // NS

_LOG2 = float(np.log(2.0))
_STEP = CUT / (NG - 1)
_COEFF = -0.5 / _STEP ** 2


def _ssp(x):
    return jax.nn.softplus(x) - _LOG2


def _sc_dist_body(pos_hbm, row_hbm, col_hbm, d2_hbm, pos_t, rowb, colb, d2b):
    wid = lax.axis_index("s") * NC + lax.axis_index("c")
    base = wid * EPW
    pltpu.sync_copy(pos_hbm, pos_t)
    pltpu.sync_copy(row_hbm.at[pl.ds(base, EPW)], rowb)
    pltpu.sync_copy(col_hbm.at[pl.ds(base, EPW)], colb)

    def body(g, carry):
        sl = pl.ds(g * 16, 16)
        r3 = rowb[sl] * 3
        c3 = colb[sl] * 3
        dx = plsc.load_gather(pos_t, [r3]) - plsc.load_gather(pos_t, [c3])
        dy = plsc.load_gather(pos_t, [r3 + 1]) - plsc.load_gather(pos_t, [c3 + 1])
        dz = plsc.load_gather(pos_t, [r3 + 2]) - plsc.load_gather(pos_t, [c3 + 2])
        d2b[sl] = dx * dx + dy * dy + dz * dz
        return carry

    lax.fori_loop(0, EPW // 16, body, 0, unroll=2)
    pltpu.sync_copy(d2b, d2_hbm.at[pl.ds(base, EPW)])


@jax.jit
def _sc_dist(pos_flat, row_p, col_p):
    mesh = plsc.VectorSubcoreMesh(core_axis_name="c", subcore_axis_name="s",
                                  num_cores=NC, num_subcores=NS)
    return pl.kernel(
        _sc_dist_body,
        out_type=jax.ShapeDtypeStruct((EP,), jnp.float32),
        mesh=mesh,
        compiler_params=pltpu.CompilerParams(needs_layout_passes=False),
        scratch_types=[
            pltpu.VMEM((N * 3,), jnp.float32),
            pltpu.VMEM((EPW,), jnp.int32),
            pltpu.VMEM((EPW,), jnp.int32),
            pltpu.VMEM((EPW,), jnp.float32),
        ],
    )(pos_flat, row_p, col_p)


def _sc_gs_body(x1_hbm, wc_hbm, row_hbm, col_hbm, agg_hbm,
                colb, rowb, rb2, xjb, wcb, shared, gsem, wsem, ssem):
    c = lax.axis_index("c")
    s = lax.axis_index("s")
    wid = s * NC + c
    ebase = wid * EPW
    r0 = s * ROWS_PER_SUB

    def stage_iblk(k):
        off = k * IBLK * CHUNK
        pltpu.sync_copy(col_hbm.at[wid, pl.ds(off, IBLK * CHUNK)],
                        colb.at[k % 2])
        pltpu.sync_copy(row_hbm.at[wid, pl.ds(off, IBLK * CHUNK)], rowb)

    def build_rb2(slot, loff):
        def ib(g, carry2):
            rb2[slot, pl.ds(g * 16, 16)] = rowb[pl.ds(loff + g * 16, 16)]
            return carry2
        lax.fori_loop(0, CHUNK // 16, ib, 0)

    def issue_gw(ch, cslot, loff, slot3, slot2):
        g = pltpu.async_copy(
            x1_hbm.at[colb.at[cslot, pl.ds(loff, CHUNK)]], xjb.at[slot3],
            gsem.at[slot3])
        w = pltpu.async_copy(
            wc_hbm.at[pl.ds(ebase + ch * CHUNK, CHUNK)], wcb.at[slot2],
            wsem.at[slot2])
        return g, w

    def wait_g(slot3):
        pltpu.make_async_copy(x1_hbm.at[pl.ds(0, CHUNK)], xjb.at[slot3],
                              gsem.at[slot3]).wait()

    def wait_w(slot2):
        pltpu.make_async_copy(wc_hbm.at[pl.ds(0, CHUNK)], wcb.at[slot2],
                              wsem.at[slot2]).wait()

    def wait_s(slot3):
        pltpu.make_async_copy(wc_hbm.at[pl.ds(0, CHUNK)], xjb.at[slot3],
                              ssem.at[slot3]).wait()

    def zbody(e, carry):
        for j in range(8):
            xjb[0, e, pl.ds(j * 16, 16)] = jnp.zeros((16,), jnp.float32)
        return carry
    lax.fori_loop(0, CHUNK, zbody, 0, unroll=2)
    for k in range(ROWS_PER_SUB // CHUNK):
        pltpu.sync_copy(xjb.at[0], shared.at[pl.ds(r0 + k * CHUNK, CHUNK)])
    plsc.subcore_barrier()

    stage_iblk(0)
    for ch in range(2):
        build_rb2(ch, ch * CHUNK)
        issue_gw(ch, 0, ch * CHUNK, ch, ch)

    def body(ch, carry):
        b3 = lax.rem(ch, 3)
        b2 = lax.rem(ch, 2)
        wait_g(b3)
        wait_w(b2)

        def mul(e, carry2):
            for j in range(8):
                sl = pl.ds(j * 16, 16)
                xjb[b3, e, sl] = xjb[b3, e, sl] * wcb[b2, e, sl]
            return carry2
        lax.fori_loop(0, CHUNK, mul, 0, unroll=4)

        pltpu.async_copy(xjb.at[b3], shared.at[rb2.at[b3]], ssem.at[b3],
                         add=True)

        for k in range(1, NCHUNK // IBLK):
            @pl.when(ch == k * IBLK - 2)
            def _():
                stage_iblk(k)

        nxt = ch + 2

        @pl.when(nxt < NCHUNK)
        def _():
            nb3 = lax.rem(nxt, 3)
            nb2 = lax.rem(nxt, 2)

            @pl.when(ch >= 1)
            def _():
                wait_s(nb3)
            loff = lax.rem(nxt, IBLK) * CHUNK
            cslot = jnp.where(lax.rem(nxt, 2 * IBLK) >= IBLK, 1, 0)
            build_rb2(nb3, loff)
            issue_gw(nxt, cslot, loff, nb3, nb2)
        return carry

    lax.fori_loop(0, NCHUNK, body, 0)
    for ch in range(NCHUNK - 3, NCHUNK):
        wait_s(ch % 3)
    plsc.subcore_barrier()
    pltpu.sync_copy(shared.at[pl.ds(r0, ROWS_PER_SUB)],
                    agg_hbm.at[c, pl.ds(r0, ROWS_PER_SUB)])


@jax.jit
def _sc_gather_scatter(x1, wc, row2d, col2d):
    mesh = plsc.VectorSubcoreMesh(core_axis_name="c", subcore_axis_name="s",
                                  num_cores=NC, num_subcores=NS)
    return pl.kernel(
        _sc_gs_body,
        out_type=jax.ShapeDtypeStruct((NC, NP, F), jnp.float32),
        mesh=mesh,
        compiler_params=pltpu.CompilerParams(needs_layout_passes=False),
        scratch_types=[
            pltpu.VMEM((2, IBLK * CHUNK), jnp.int32),
            pltpu.VMEM((IBLK * CHUNK,), jnp.int32),
            pltpu.VMEM((3, CHUNK), jnp.int32),
            pltpu.VMEM((3, CHUNK, F), jnp.float32),
            pltpu.VMEM((2, CHUNK, F), jnp.float32),
            pltpu.VMEM_SHARED((NP, F), jnp.float32),
            pltpu.SemaphoreType.DMA((3,)),
            pltpu.SemaphoreType.DMA((2,)),
            pltpu.SemaphoreType.DMA((3,)),
        ],
    )(x1, wc, row2d, col2d)


def _tc_filter_kernel(d2_ref, w1_ref, b1_ref, w2_ref, b2_ref, wc_ref):
    i = pl.program_id(0)
    d2 = d2_ref[...]
    d = jnp.sqrt(d2 + 1e-12)
    j = lax.broadcasted_iota(jnp.int32, (NB, 64), 1).astype(jnp.float32)
    rbf = jnp.exp(_COEFF * (d - j * _STEP) ** 2)
    t = _ssp(jnp.dot(rbf, w1_ref[...], preferred_element_type=jnp.float32)
             + b1_ref[...])
    W = jnp.dot(t, w2_ref[...], preferred_element_type=jnp.float32) + b2_ref[...]
    C = 0.5 * (jnp.cos(d * (np.pi / CUT)) + 1.0)
    C = jnp.where(d < CUT, C, 0.0)
    eidx = i * NB + lax.broadcasted_iota(jnp.int32, (NB, 1), 0)
    C = jnp.where(eidx < E, C, 0.0)
    wc_ref[...] = W * C


@jax.jit
def _tc_filter(d2c, w1p, b1, w2, b2):
    return pl.pallas_call(
        _tc_filter_kernel,
        grid=(EP // NB,),
        in_specs=[
            pl.BlockSpec((NB, 1), lambda i: (i, 0)),
            pl.BlockSpec((64, F), lambda i: (0, 0)),
            pl.BlockSpec((1, F), lambda i: (0, 0)),
            pl.BlockSpec((F, F), lambda i: (0, 0)),
            pl.BlockSpec((1, F), lambda i: (0, 0)),
        ],
        out_specs=pl.BlockSpec((NB, F), lambda i: (i, 0)),
        out_shape=jax.ShapeDtypeStruct((EP, F), jnp.float32),
    )(d2c, w1p, b1, w2, b2)


def _tc_prologue_kernel(az_ref, emb_ref, l1_ref, h_ref, x1_ref):
    az = az_ref[...]
    j = lax.broadcasted_iota(jnp.int32, (NB, 128), 1).astype(jnp.float32)
    oh = (az == j).astype(jnp.float32)
    h = jnp.dot(oh, emb_ref[...], preferred_element_type=jnp.float32)
    h_ref[...] = h
    x1_ref[...] = jnp.dot(h, l1_ref[...], preferred_element_type=jnp.float32)


@jax.jit
def _tc_prologue(azf, embp, l1):
    return pl.pallas_call(
        _tc_prologue_kernel,
        grid=(NP // NB,),
        in_specs=[
            pl.BlockSpec((NB, 1), lambda i: (i, 0)),
            pl.BlockSpec((128, H), lambda i: (0, 0)),
            pl.BlockSpec((H, F), lambda i: (0, 0)),
        ],
        out_specs=[pl.BlockSpec((NB, H), lambda i: (i, 0)),
                   pl.BlockSpec((NB, F), lambda i: (i, 0))],
        out_shape=[jax.ShapeDtypeStruct((NP, H), jnp.float32),
                   jax.ShapeDtypeStruct((NP, F), jnp.float32)],
    )(azf, embp, l1)


def _tc_node_kernel(a_ref, h_ref, w2_ref, b2_ref, w3_ref, b3_ref, l1_ref,
                    hn_ref, x1_ref):
    i = pl.program_id(0)
    agg = a_ref[0] + a_ref[1]
    x = _ssp(jnp.dot(agg, w2_ref[...], preferred_element_type=jnp.float32)
             + b2_ref[...])
    x = jnp.dot(x, w3_ref[...], preferred_element_type=jnp.float32) + b3_ref[...]
    ridx = i * NB + lax.broadcasted_iota(jnp.int32, (NB, 1), 0)
    x = jnp.where(ridx < N, x, 0.0)
    hn = h_ref[...] + x
    hn_ref[...] = hn
    x1_ref[...] = jnp.dot(hn, l1_ref[...], preferred_element_type=jnp.float32)


@jax.jit
def _tc_node_update(aggp, h, w2, b2, w3, b3, l1n):
    return pl.pallas_call(
        _tc_node_kernel,
        grid=(NP // NB,),
        in_specs=[
            pl.BlockSpec((2, NB, F), lambda i: (0, i, 0)),
            pl.BlockSpec((NB, H), lambda i: (i, 0)),
            pl.BlockSpec((F, H), lambda i: (0, 0)),
            pl.BlockSpec((1, H), lambda i: (0, 0)),
            pl.BlockSpec((H, H), lambda i: (0, 0)),
            pl.BlockSpec((1, H), lambda i: (0, 0)),
            pl.BlockSpec((H, F), lambda i: (0, 0)),
        ],
        out_specs=[pl.BlockSpec((NB, H), lambda i: (i, 0)),
                   pl.BlockSpec((NB, F), lambda i: (i, 0))],
        out_shape=[jax.ShapeDtypeStruct((NP, H), jnp.float32),
                   jax.ShapeDtypeStruct((NP, F), jnp.float32)],
    )(aggp, h, w2, b2, w3, b3, l1n)


def _tc_film_kernel(did_ref, de_ref, w1_ref, b1_ref, w2_ref, b2_ref,
                    gw_ref, gb_ref, bw_ref, bb_ref, gam_ref, bet_ref):
    did = did_ref[...]
    j = lax.broadcasted_iota(jnp.int32, (G, 8), 1).astype(jnp.float32)
    oh = (did == j).astype(jnp.float32)
    de = jnp.dot(oh, de_ref[...], preferred_element_type=jnp.float32)
    fc = jax.nn.relu(jnp.dot(de, w1_ref[...], preferred_element_type=jnp.float32)
                     + b1_ref[...])
    fc = jnp.dot(fc, w2_ref[...], preferred_element_type=jnp.float32) + b2_ref[...]
    gam_ref[...] = jnp.dot(fc, gw_ref[...],
                           preferred_element_type=jnp.float32) + gb_ref[...]
    bet_ref[...] = jnp.dot(fc, bw_ref[...],
                           preferred_element_type=jnp.float32) + bb_ref[...]


@jax.jit
def _tc_film(didf, dep, fp1_w, fp1_b, fp2_w, fp2_b, gam_w, gam_b, bet_w, bet_b):
    full = lambda *s: pl.BlockSpec(s, lambda: tuple(0 for _ in s))
    return pl.pallas_call(
        _tc_film_kernel,
        in_specs=[full(G, 1), full(8, 64), full(64, 128), full(1, 128),
                  full(128, 128), full(1, 128), full(128, H), full(1, H),
                  full(128, H), full(1, H)],
        out_specs=[full(G, H), full(G, H)],
        out_shape=[jax.ShapeDtypeStruct((G, H), jnp.float32),
                   jax.ShapeDtypeStruct((G, H), jnp.float32)],
    )(didf, dep, fp1_w, fp1_b, fp2_w, fp2_b, gam_w, gam_b, bet_w, bet_b)


def _tc_final_kernel(h_ref, b_ref, gam_ref, bet_ref, w1_ref, b1_ref,
                     w2_ref, b2_ref, out_ref):
    i = pl.program_id(0)
    bf = b_ref[...]
    j = lax.broadcasted_iota(jnp.int32, (NB, G), 1).astype(jnp.float32)
    oh = (bf == j).astype(jnp.float32)
    gam_n = jnp.dot(oh, gam_ref[...], preferred_element_type=jnp.float32)
    bet_n = jnp.dot(oh, bet_ref[...], preferred_element_type=jnp.float32)
    hp = gam_n * h_ref[...] + bet_n
    t = _ssp(jnp.dot(hp, w1_ref[...], preferred_element_type=jnp.float32)
             + b1_ref[...])
    o = jnp.dot(t, w2_ref[...], preferred_element_type=jnp.float32) + b2_ref[...]
    part = lax.dot_general(oh, o, (((0,), (0,)), ((), ())),
                           preferred_element_type=jnp.float32)

    @pl.when(i == 0)
    def _():
        out_ref[...] = jnp.zeros_like(out_ref)

    out_ref[...] += part


@jax.jit
def _tc_final(h, batchf, gamma, beta, out1_w, out1_b, out2_wp, out2_bp):
    return pl.pallas_call(
        _tc_final_kernel,
        grid=(NP // NB,),
        in_specs=[
            pl.BlockSpec((NB, H), lambda i: (i, 0)),
            pl.BlockSpec((NB, 1), lambda i: (i, 0)),
            pl.BlockSpec((G, H), lambda i: (0, 0)),
            pl.BlockSpec((G, H), lambda i: (0, 0)),
            pl.BlockSpec((H, H // 2), lambda i: (0, 0)),
            pl.BlockSpec((1, H // 2), lambda i: (0, 0)),
            pl.BlockSpec((H // 2, 8), lambda i: (0, 0)),
            pl.BlockSpec((1, 8), lambda i: (0, 0)),
        ],
        out_specs=pl.BlockSpec((G, 8), lambda i: (0, 0)),
        out_shape=jax.ShapeDtypeStruct((G, 8), jnp.float32),
    )(h, batchf, gamma, beta, out1_w, out1_b, out2_wp, out2_bp)


def kernel(pos, atomic_numbers, batch, edge_index, domain_ids, emb, mlp_w1,
           mlp_b1, mlp_w2, mlp_b2, lin1_w, lin2_w, lin2_b, lin3_w, lin3_b,
           out1_w, out1_b, out2_w, out2_b, dom_emb, fp1_w, fp1_b, fp2_w,
           fp2_b, gam_w, gam_b, bet_w, bet_b):
    npad = EP - E
    pad_idx = (jnp.arange(npad, dtype=jnp.int32) * 7) % N
    row_p = jnp.concatenate([edge_index[0].astype(jnp.int32), pad_idx])
    col_p = jnp.concatenate([edge_index[1].astype(jnp.int32), pad_idx])
    row2d = row_p.reshape(NW, EPW)
    col2d = col_p.reshape(NW, EPW)
    pos_flat = pos.reshape(-1)

    azf = jnp.pad(atomic_numbers.astype(jnp.float32), (0, NP - N),
                  constant_values=float(ZMAX)).reshape(NP, 1)
    batchf = jnp.pad(batch.astype(jnp.float32), (0, NP - N),
                     constant_values=float(G)).reshape(NP, 1)
    didf = domain_ids.astype(jnp.float32).reshape(G, 1)

    w1p = jnp.pad(mlp_w1, ((0, 0), (0, 64 - NG), (0, 0)))
    embp = jnp.pad(emb, ((0, 128 - ZMAX), (0, 0)))
    dep = jnp.pad(dom_emb, ((0, 3), (0, 0)))
    out2_wp = jnp.pad(out2_w, ((0, 0), (0, 7)))
    out2_bp = jnp.pad(out2_b, (0, 7)).reshape(1, 8)
    r2 = lambda v: v.reshape(1, -1)

    d2 = _sc_dist(pos_flat, row_p, col_p)
    d2c = d2.reshape(EP, 1)
    h, x1 = _tc_prologue(azf, embp, lin1_w[0])
    for i in range(NI):
        wc = _tc_filter(d2c, w1p[i], r2(mlp_b1[i]), mlp_w2[i], r2(mlp_b2[i]))
        aggp = _sc_gather_scatter(x1, wc, row2d, col2d)
        h, x1 = _tc_node_update(aggp, h, lin2_w[i], r2(lin2_b[i]),
                                lin3_w[i], r2(lin3_b[i]),
                                lin1_w[(i + 1) % NI])
    gamma, beta = _tc_film(didf, dep, fp1_w, r2(fp1_b), fp2_w, r2(fp2_b),
                           gam_w, r2(gam_b), bet_w, r2(bet_b))
    eng = _tc_final(h, batchf, gamma, beta, out1_w, r2(out1_b),
                    out2_wp, out2_bp)
    return eng[:, :1]


# speedup vs baseline: 2.7010x; 1.3758x over previous
import functools

import jax
import jax.numpy as jnp
import numpy as np
from jax import lax
from jax.experimental import pallas as pl
from jax.experimental.pallas import tpu as pltpu
from jax.experimental.pallas import tpu_sc as plsc

N = 10000
E = 320000
H = 128
F = 128
NG = 50
NI = 6
G = 128
CUT = 10.0
ZMAX = 100

NC = 2
NS = 16
NW = NC * NS
CHUNK = 64
NCHUNK = 160
IBLK = 32
EPW = NCHUNK * CHUNK
EP = NW * EPW
NP = 10240
NB = 512
ROWS_PER_SUB = NP // NS

_LOG2 = float(np.log(2.0))
_STEP = CUT / (NG - 1)
_COEFF = -0.5 / _STEP ** 2


def _ssp(x):
    return jax.nn.softplus(x) - _LOG2


def _sc_dist_body(pos_hbm, row_hbm, col_hbm, d2_hbm, pos_t, rowb, colb, d2b):
    wid = lax.axis_index("s") * NC + lax.axis_index("c")
    base = wid * EPW
    pltpu.sync_copy(pos_hbm, pos_t)
    pltpu.sync_copy(row_hbm.at[pl.ds(base, EPW)], rowb)
    pltpu.sync_copy(col_hbm.at[pl.ds(base, EPW)], colb)

    def body(g, carry):
        sl = pl.ds(g * 16, 16)
        r3 = rowb[sl] * 3
        c3 = colb[sl] * 3
        dx = plsc.load_gather(pos_t, [r3]) - plsc.load_gather(pos_t, [c3])
        dy = plsc.load_gather(pos_t, [r3 + 1]) - plsc.load_gather(pos_t, [c3 + 1])
        dz = plsc.load_gather(pos_t, [r3 + 2]) - plsc.load_gather(pos_t, [c3 + 2])
        d2b[sl] = dx * dx + dy * dy + dz * dz
        return carry

    lax.fori_loop(0, EPW // 16, body, 0, unroll=2)
    pltpu.sync_copy(d2b, d2_hbm.at[pl.ds(base, EPW)])


@jax.jit
def _sc_dist(pos_flat, row_p, col_p):
    mesh = plsc.VectorSubcoreMesh(core_axis_name="c", subcore_axis_name="s",
                                  num_cores=NC, num_subcores=NS)
    return pl.kernel(
        _sc_dist_body,
        out_type=jax.ShapeDtypeStruct((EP,), jnp.float32),
        mesh=mesh,
        compiler_params=pltpu.CompilerParams(needs_layout_passes=False),
        scratch_types=[
            pltpu.VMEM((N * 3,), jnp.float32),
            pltpu.VMEM((EPW,), jnp.int32),
            pltpu.VMEM((EPW,), jnp.int32),
            pltpu.VMEM((EPW,), jnp.float32),
        ],
    )(pos_flat, row_p, col_p)


def _sc_gs_body(x1_hbm, wc_hbm, row_hbm, col_hbm, agg_hbm,
                colb, rowb, rb2, xjb, wcb, shared, gsem, wsem, ssem):
    c = lax.axis_index("c")
    s = lax.axis_index("s")
    wid = s * NC + c
    ebase = wid * EPW
    r0 = s * ROWS_PER_SUB

    def stage_iblk(k):
        off = k * IBLK * CHUNK
        pltpu.sync_copy(col_hbm.at[wid, pl.ds(off, IBLK * CHUNK)],
                        colb.at[k % 2])
        pltpu.sync_copy(row_hbm.at[wid, pl.ds(off, IBLK * CHUNK)], rowb)

    def build_rb2(slot, loff):
        def ib(g, carry2):
            rb2[slot, pl.ds(g * 16, 16)] = rowb[pl.ds(loff + g * 16, 16)]
            return carry2
        lax.fori_loop(0, CHUNK // 16, ib, 0)

    def issue_gw(ch, cslot, loff, slot3, slot2):
        g = pltpu.async_copy(
            x1_hbm.at[colb.at[cslot, pl.ds(loff, CHUNK)]], xjb.at[slot3],
            gsem.at[slot3])
        w = pltpu.async_copy(
            wc_hbm.at[pl.ds(ebase + ch * CHUNK, CHUNK)], wcb.at[slot2],
            wsem.at[slot2])
        return g, w

    def wait_g(slot3):
        pltpu.make_async_copy(x1_hbm.at[pl.ds(0, CHUNK)], xjb.at[slot3],
                              gsem.at[slot3]).wait()

    def wait_w(slot2):
        pltpu.make_async_copy(wc_hbm.at[pl.ds(0, CHUNK)], wcb.at[slot2],
                              wsem.at[slot2]).wait()

    def wait_s(slot3):
        pltpu.make_async_copy(wc_hbm.at[pl.ds(0, CHUNK)], xjb.at[slot3],
                              ssem.at[slot3]).wait()

    def zbody(e, carry):
        for j in range(8):
            xjb[0, e, pl.ds(j * 16, 16)] = jnp.zeros((16,), jnp.float32)
        return carry
    lax.fori_loop(0, CHUNK, zbody, 0, unroll=2)
    for k in range(ROWS_PER_SUB // CHUNK):
        pltpu.sync_copy(xjb.at[0], shared.at[pl.ds(r0 + k * CHUNK, CHUNK)])
    plsc.subcore_barrier()

    stage_iblk(0)
    for ch in range(2):
        build_rb2(ch, ch * CHUNK)
        issue_gw(ch, 0, ch * CHUNK, ch, ch)

    def body(ch, carry):
        b3 = lax.rem(ch, 3)
        b2 = lax.rem(ch, 2)
        wait_g(b3)
        wait_w(b2)

        def mul(e, carry2):
            for j in range(8):
                sl = pl.ds(j * 16, 16)
                xjb[b3, e, sl] = xjb[b3, e, sl] * wcb[b2, e, sl]
            return carry2
        lax.fori_loop(0, CHUNK, mul, 0, unroll=4)

        pltpu.async_copy(xjb.at[b3], shared.at[rb2.at[b3]], ssem.at[b3],
                         add=True)

        for k in range(1, NCHUNK // IBLK):
            @pl.when(ch == k * IBLK - 2)
            def _():
                stage_iblk(k)

        nxt = ch + 2

        @pl.when(nxt < NCHUNK)
        def _():
            nb3 = lax.rem(nxt, 3)
            nb2 = lax.rem(nxt, 2)

            @pl.when(ch >= 1)
            def _():
                wait_s(nb3)
            loff = lax.rem(nxt, IBLK) * CHUNK
            cslot = jnp.where(lax.rem(nxt, 2 * IBLK) >= IBLK, 1, 0)
            build_rb2(nb3, loff)
            issue_gw(nxt, cslot, loff, nb3, nb2)
        return carry

    lax.fori_loop(0, NCHUNK, body, 0)
    for ch in range(NCHUNK - 3, NCHUNK):
        wait_s(ch % 3)
    plsc.subcore_barrier()
    pltpu.sync_copy(shared.at[pl.ds(r0, ROWS_PER_SUB)],
                    agg_hbm.at[c, pl.ds(r0, ROWS_PER_SUB)])


@jax.jit
def _sc_gather_scatter(x1, wc, row2d, col2d):
    mesh = plsc.VectorSubcoreMesh(core_axis_name="c", subcore_axis_name="s",
                                  num_cores=NC, num_subcores=NS)
    return pl.kernel(
        _sc_gs_body,
        out_type=jax.ShapeDtypeStruct((NC, NP, F), jnp.float32),
        mesh=mesh,
        compiler_params=pltpu.CompilerParams(needs_layout_passes=False),
        scratch_types=[
            pltpu.VMEM((2, IBLK * CHUNK), jnp.int32),
            pltpu.VMEM((IBLK * CHUNK,), jnp.int32),
            pltpu.VMEM((3, CHUNK), jnp.int32),
            pltpu.VMEM((3, CHUNK, F), jnp.float32),
            pltpu.VMEM((2, CHUNK, F), jnp.float32),
            pltpu.VMEM_SHARED((NP, F), jnp.float32),
            pltpu.SemaphoreType.DMA((3,)),
            pltpu.SemaphoreType.DMA((2,)),
            pltpu.SemaphoreType.DMA((3,)),
        ],
    )(x1, wc, row2d, col2d)


def _tc_edgeprep_kernel(d2_ref, d_ref, c_ref):
    i = pl.program_id(0)
    d = jnp.sqrt(d2_ref[...] + 1e-12)
    C = 0.5 * (jnp.cos(d * (np.pi / CUT)) + 1.0)
    C = jnp.where(d < CUT, C, 0.0)
    eidx = (i * 8192
            + lax.broadcasted_iota(jnp.int32, (64, 128), 0) * 128
            + lax.broadcasted_iota(jnp.int32, (64, 128), 1))
    C = jnp.where(eidx < E, C, 0.0)
    d_ref[...] = d
    c_ref[...] = C


@jax.jit
def _tc_edgeprep(d2r):
    return pl.pallas_call(
        _tc_edgeprep_kernel,
        grid=(EP // 8192,),
        in_specs=[pl.BlockSpec((64, 128), lambda i: (i, 0))],
        out_specs=[pl.BlockSpec((64, 128), lambda i: (i, 0)),
                   pl.BlockSpec((64, 128), lambda i: (i, 0))],
        out_shape=[jax.ShapeDtypeStruct((EP // 128, 128), jnp.float32),
                   jax.ShapeDtypeStruct((EP // 128, 128), jnp.float32)],
    )(d2r)


def _tc_filter_kernel(d_ref, c_ref, w1_ref, b1_ref, w2_ref, b2_ref, wc_ref):
    d = d_ref[...]
    j = lax.broadcasted_iota(jnp.int32, (NB, 64), 1).astype(jnp.float32)
    rbf = jnp.exp(_COEFF * (d - j * _STEP) ** 2)
    t = _ssp(jnp.dot(rbf, w1_ref[...], preferred_element_type=jnp.float32)
             + b1_ref[...])
    W = jnp.dot(t, w2_ref[...], preferred_element_type=jnp.float32) + b2_ref[...]
    wc_ref[...] = W * c_ref[...]


@jax.jit
def _tc_filter(d_c, c_c, w1p, b1, w2, b2):
    return pl.pallas_call(
        _tc_filter_kernel,
        grid=(EP // NB,),
        in_specs=[
            pl.BlockSpec((NB, 1), lambda i: (i, 0)),
            pl.BlockSpec((NB, 1), lambda i: (i, 0)),
            pl.BlockSpec((64, F), lambda i: (0, 0)),
            pl.BlockSpec((1, F), lambda i: (0, 0)),
            pl.BlockSpec((F, F), lambda i: (0, 0)),
            pl.BlockSpec((1, F), lambda i: (0, 0)),
        ],
        out_specs=pl.BlockSpec((NB, F), lambda i: (i, 0)),
        out_shape=jax.ShapeDtypeStruct((EP, F), jnp.float32),
    )(d_c, c_c, w1p, b1, w2, b2)


def _tc_prologue_kernel(az_ref, emb_ref, l1_ref, h_ref, x1_ref):
    az = az_ref[...]
    j = lax.broadcasted_iota(jnp.int32, (NB, 128), 1).astype(jnp.float32)
    oh = (az == j).astype(jnp.float32)
    h = jnp.dot(oh, emb_ref[...], preferred_element_type=jnp.float32)
    h_ref[...] = h
    x1_ref[...] = jnp.dot(h, l1_ref[...], preferred_element_type=jnp.float32)


@jax.jit
def _tc_prologue(azf, embp, l1):
    return pl.pallas_call(
        _tc_prologue_kernel,
        grid=(NP // NB,),
        in_specs=[
            pl.BlockSpec((NB, 1), lambda i: (i, 0)),
            pl.BlockSpec((128, H), lambda i: (0, 0)),
            pl.BlockSpec((H, F), lambda i: (0, 0)),
        ],
        out_specs=[pl.BlockSpec((NB, H), lambda i: (i, 0)),
                   pl.BlockSpec((NB, F), lambda i: (i, 0))],
        out_shape=[jax.ShapeDtypeStruct((NP, H), jnp.float32),
                   jax.ShapeDtypeStruct((NP, F), jnp.float32)],
    )(azf, embp, l1)


def _tc_node_kernel(a_ref, h_ref, w2_ref, b2_ref, w3_ref, b3_ref, l1_ref,
                    hn_ref, x1_ref):
    i = pl.program_id(0)
    agg = a_ref[0] + a_ref[1]
    x = _ssp(jnp.dot(agg, w2_ref[...], preferred_element_type=jnp.float32)
             + b2_ref[...])
    x = jnp.dot(x, w3_ref[...], preferred_element_type=jnp.float32) + b3_ref[...]
    ridx = i * NB + lax.broadcasted_iota(jnp.int32, (NB, 1), 0)
    x = jnp.where(ridx < N, x, 0.0)
    hn = h_ref[...] + x
    hn_ref[...] = hn
    x1_ref[...] = jnp.dot(hn, l1_ref[...], preferred_element_type=jnp.float32)


@jax.jit
def _tc_node_update(aggp, h, w2, b2, w3, b3, l1n):
    return pl.pallas_call(
        _tc_node_kernel,
        grid=(NP // NB,),
        in_specs=[
            pl.BlockSpec((2, NB, F), lambda i: (0, i, 0)),
            pl.BlockSpec((NB, H), lambda i: (i, 0)),
            pl.BlockSpec((F, H), lambda i: (0, 0)),
            pl.BlockSpec((1, H), lambda i: (0, 0)),
            pl.BlockSpec((H, H), lambda i: (0, 0)),
            pl.BlockSpec((1, H), lambda i: (0, 0)),
            pl.BlockSpec((H, F), lambda i: (0, 0)),
        ],
        out_specs=[pl.BlockSpec((NB, H), lambda i: (i, 0)),
                   pl.BlockSpec((NB, F), lambda i: (i, 0))],
        out_shape=[jax.ShapeDtypeStruct((NP, H), jnp.float32),
                   jax.ShapeDtypeStruct((NP, F), jnp.float32)],
    )(aggp, h, w2, b2, w3, b3, l1n)


def _tc_film_kernel(did_ref, de_ref, w1_ref, b1_ref, w2_ref, b2_ref,
                    gw_ref, gb_ref, bw_ref, bb_ref, gam_ref, bet_ref):
    did = did_ref[...]
    j = lax.broadcasted_iota(jnp.int32, (G, 8), 1).astype(jnp.float32)
    oh = (did == j).astype(jnp.float32)
    de = jnp.dot(oh, de_ref[...], preferred_element_type=jnp.float32)
    fc = jax.nn.relu(jnp.dot(de, w1_ref[...], preferred_element_type=jnp.float32)
                     + b1_ref[...])
    fc = jnp.dot(fc, w2_ref[...], preferred_element_type=jnp.float32) + b2_ref[...]
    gam_ref[...] = jnp.dot(fc, gw_ref[...],
                           preferred_element_type=jnp.float32) + gb_ref[...]
    bet_ref[...] = jnp.dot(fc, bw_ref[...],
                           preferred_element_type=jnp.float32) + bb_ref[...]


@jax.jit
def _tc_film(didf, dep, fp1_w, fp1_b, fp2_w, fp2_b, gam_w, gam_b, bet_w, bet_b):
    full = lambda *s: pl.BlockSpec(s, lambda: tuple(0 for _ in s))
    return pl.pallas_call(
        _tc_film_kernel,
        in_specs=[full(G, 1), full(8, 64), full(64, 128), full(1, 128),
                  full(128, 128), full(1, 128), full(128, H), full(1, H),
                  full(128, H), full(1, H)],
        out_specs=[full(G, H), full(G, H)],
        out_shape=[jax.ShapeDtypeStruct((G, H), jnp.float32),
                   jax.ShapeDtypeStruct((G, H), jnp.float32)],
    )(didf, dep, fp1_w, fp1_b, fp2_w, fp2_b, gam_w, gam_b, bet_w, bet_b)


def _tc_final_kernel(h_ref, b_ref, gam_ref, bet_ref, w1_ref, b1_ref,
                     w2_ref, b2_ref, out_ref):
    i = pl.program_id(0)
    bf = b_ref[...]
    j = lax.broadcasted_iota(jnp.int32, (NB, G), 1).astype(jnp.float32)
    oh = (bf == j).astype(jnp.float32)
    gam_n = jnp.dot(oh, gam_ref[...], preferred_element_type=jnp.float32)
    bet_n = jnp.dot(oh, bet_ref[...], preferred_element_type=jnp.float32)
    hp = gam_n * h_ref[...] + bet_n
    t = _ssp(jnp.dot(hp, w1_ref[...], preferred_element_type=jnp.float32)
             + b1_ref[...])
    o = jnp.dot(t, w2_ref[...], preferred_element_type=jnp.float32) + b2_ref[...]
    part = lax.dot_general(oh, o, (((0,), (0,)), ((), ())),
                           preferred_element_type=jnp.float32)

    @pl.when(i == 0)
    def _():
        out_ref[...] = jnp.zeros_like(out_ref)

    out_ref[...] += part


@jax.jit
def _tc_final(h, batchf, gamma, beta, out1_w, out1_b, out2_wp, out2_bp):
    return pl.pallas_call(
        _tc_final_kernel,
        grid=(NP // NB,),
        in_specs=[
            pl.BlockSpec((NB, H), lambda i: (i, 0)),
            pl.BlockSpec((NB, 1), lambda i: (i, 0)),
            pl.BlockSpec((G, H), lambda i: (0, 0)),
            pl.BlockSpec((G, H), lambda i: (0, 0)),
            pl.BlockSpec((H, H // 2), lambda i: (0, 0)),
            pl.BlockSpec((1, H // 2), lambda i: (0, 0)),
            pl.BlockSpec((H // 2, 8), lambda i: (0, 0)),
            pl.BlockSpec((1, 8), lambda i: (0, 0)),
        ],
        out_specs=pl.BlockSpec((G, 8), lambda i: (0, 0)),
        out_shape=jax.ShapeDtypeStruct((G, 8), jnp.float32),
    )(h, batchf, gamma, beta, out1_w, out1_b, out2_wp, out2_bp)


def kernel(pos, atomic_numbers, batch, edge_index, domain_ids, emb, mlp_w1,
           mlp_b1, mlp_w2, mlp_b2, lin1_w, lin2_w, lin2_b, lin3_w, lin3_b,
           out1_w, out1_b, out2_w, out2_b, dom_emb, fp1_w, fp1_b, fp2_w,
           fp2_b, gam_w, gam_b, bet_w, bet_b):
    npad = EP - E
    pad_idx = (jnp.arange(npad, dtype=jnp.int32) * 7) % N
    row_p = jnp.concatenate([edge_index[0].astype(jnp.int32), pad_idx])
    col_p = jnp.concatenate([edge_index[1].astype(jnp.int32), pad_idx])
    row2d = row_p.reshape(NW, EPW)
    col2d = col_p.reshape(NW, EPW)
    pos_flat = pos.reshape(-1)

    azf = jnp.pad(atomic_numbers.astype(jnp.float32), (0, NP - N),
                  constant_values=float(ZMAX)).reshape(NP, 1)
    batchf = jnp.pad(batch.astype(jnp.float32), (0, NP - N),
                     constant_values=float(G)).reshape(NP, 1)
    didf = domain_ids.astype(jnp.float32).reshape(G, 1)

    w1p = jnp.pad(mlp_w1, ((0, 0), (0, 64 - NG), (0, 0)))
    embp = jnp.pad(emb, ((0, 128 - ZMAX), (0, 0)))
    dep = jnp.pad(dom_emb, ((0, 3), (0, 0)))
    out2_wp = jnp.pad(out2_w, ((0, 0), (0, 7)))
    out2_bp = jnp.pad(out2_b, (0, 7)).reshape(1, 8)
    r2 = lambda v: v.reshape(1, -1)

    d2 = _sc_dist(pos_flat, row_p, col_p)
    dd, cc = _tc_edgeprep(d2.reshape(EP // 128, 128))
    d_c = dd.reshape(EP, 1)
    c_c = cc.reshape(EP, 1)
    h, x1 = _tc_prologue(azf, embp, lin1_w[0])
    for i in range(NI):
        wc = _tc_filter(d_c, c_c, w1p[i], r2(mlp_b1[i]), mlp_w2[i],
                        r2(mlp_b2[i]))
        aggp = _sc_gather_scatter(x1, wc, row2d, col2d)
        h, x1 = _tc_node_update(aggp, h, lin2_w[i], r2(lin2_b[i]),
                                lin3_w[i], r2(lin3_b[i]),
                                lin1_w[(i + 1) % NI])
    gamma, beta = _tc_film(didf, dep, fp1_w, r2(fp1_b), fp2_w, r2(fp2_b),
                           gam_w, r2(gam_b), bet_w, r2(bet_b))
    eng = _tc_final(h, batchf, gamma, beta, out1_w, r2(out1_b),
                    out2_wp, out2_bp)
    return eng[:, :1]


# speedup vs baseline: 2.7746x; 1.0273x over previous
import functools

import jax
import jax.numpy as jnp
import numpy as np
from jax import lax
from jax.experimental import pallas as pl
from jax.experimental.pallas import tpu as pltpu
from jax.experimental.pallas import tpu_sc as plsc

N = 10000
E = 320000
H = 128
F = 128
NG = 50
NI = 6
G = 128
CUT = 10.0
ZMAX = 100

NC = 2
NS = 16
NW = NC * NS
CHUNK = 64
NCHUNK = 160
IBLK = 32
EPW = NCHUNK * CHUNK
EP = NW * EPW
NP = 10240
NB = 512
ROWS_PER_SUB = NP // NS

_LOG2 = float(np.log(2.0))
_STEP = CUT / (NG - 1)
_COEFF = -0.5 / _STEP ** 2


def _ssp(x):
    return jnp.where(x > 30.0, x - _LOG2,
                     _LOG2 * jnp.log2(0.5 + 0.5 * jnp.exp(x)))


def _sc_dist_body(pos_hbm, row_hbm, col_hbm, d2_hbm, pos_t, rowb, colb, d2b):
    wid = lax.axis_index("s") * NC + lax.axis_index("c")
    base = wid * EPW
    pltpu.sync_copy(pos_hbm, pos_t)
    pltpu.sync_copy(row_hbm.at[pl.ds(base, EPW)], rowb)
    pltpu.sync_copy(col_hbm.at[pl.ds(base, EPW)], colb)

    def body(g, carry):
        sl = pl.ds(g * 16, 16)
        r3 = rowb[sl] * 3
        c3 = colb[sl] * 3
        dx = plsc.load_gather(pos_t, [r3]) - plsc.load_gather(pos_t, [c3])
        dy = plsc.load_gather(pos_t, [r3 + 1]) - plsc.load_gather(pos_t, [c3 + 1])
        dz = plsc.load_gather(pos_t, [r3 + 2]) - plsc.load_gather(pos_t, [c3 + 2])
        d2b[sl] = dx * dx + dy * dy + dz * dz
        return carry

    lax.fori_loop(0, EPW // 16, body, 0, unroll=2)
    pltpu.sync_copy(d2b, d2_hbm.at[pl.ds(base, EPW)])


@jax.jit
def _sc_dist(pos_flat, row_p, col_p):
    mesh = plsc.VectorSubcoreMesh(core_axis_name="c", subcore_axis_name="s",
                                  num_cores=NC, num_subcores=NS)
    return pl.kernel(
        _sc_dist_body,
        out_type=jax.ShapeDtypeStruct((EP,), jnp.float32),
        mesh=mesh,
        compiler_params=pltpu.CompilerParams(needs_layout_passes=False),
        scratch_types=[
            pltpu.VMEM((N * 3,), jnp.float32),
            pltpu.VMEM((EPW,), jnp.int32),
            pltpu.VMEM((EPW,), jnp.int32),
            pltpu.VMEM((EPW,), jnp.float32),
        ],
    )(pos_flat, row_p, col_p)


def _sc_gs_body(x1_hbm, wc_hbm, row_hbm, col_hbm, agg_hbm,
                colb, rowb, rb2, xjb, wcb, shared, gsem, wsem, ssem):
    c = lax.axis_index("c")
    s = lax.axis_index("s")
    wid = s * NC + c
    ebase = wid * EPW
    r0 = s * ROWS_PER_SUB

    def stage_iblk(k):
        off = k * IBLK * CHUNK
        pltpu.sync_copy(col_hbm.at[wid, pl.ds(off, IBLK * CHUNK)],
                        colb.at[k % 2])
        pltpu.sync_copy(row_hbm.at[wid, pl.ds(off, IBLK * CHUNK)], rowb)

    def build_rb2(slot, loff):
        def ib(g, carry2):
            rb2[slot, pl.ds(g * 16, 16)] = rowb[pl.ds(loff + g * 16, 16)]
            return carry2
        lax.fori_loop(0, CHUNK // 16, ib, 0)

    def issue_gw(ch, cslot, loff, slot3, slot2):
        g = pltpu.async_copy(
            x1_hbm.at[colb.at[cslot, pl.ds(loff, CHUNK)]], xjb.at[slot3],
            gsem.at[slot3])
        w = pltpu.async_copy(
            wc_hbm.at[pl.ds(ebase + ch * CHUNK, CHUNK)], wcb.at[slot2],
            wsem.at[slot2])
        return g, w

    def wait_g(slot3):
        pltpu.make_async_copy(x1_hbm.at[pl.ds(0, CHUNK)], xjb.at[slot3],
                              gsem.at[slot3]).wait()

    def wait_w(slot2):
        pltpu.make_async_copy(wc_hbm.at[pl.ds(0, CHUNK)], wcb.at[slot2],
                              wsem.at[slot2]).wait()

    def wait_s(slot3):
        pltpu.make_async_copy(wc_hbm.at[pl.ds(0, CHUNK)], xjb.at[slot3],
                              ssem.at[slot3]).wait()

    def zbody(e, carry):
        for j in range(8):
            xjb[0, e, pl.ds(j * 16, 16)] = jnp.zeros((16,), jnp.float32)
        return carry
    lax.fori_loop(0, CHUNK, zbody, 0, unroll=2)
    for k in range(ROWS_PER_SUB // CHUNK):
        pltpu.sync_copy(xjb.at[0], shared.at[pl.ds(r0 + k * CHUNK, CHUNK)])
    plsc.subcore_barrier()

    stage_iblk(0)
    for ch in range(2):
        build_rb2(ch, ch * CHUNK)
        issue_gw(ch, 0, ch * CHUNK, ch, ch)

    def body(ch, carry):
        b3 = lax.rem(ch, 3)
        b2 = lax.rem(ch, 2)
        wait_g(b3)
        wait_w(b2)

        def mul(e, carry2):
            for j in range(8):
                sl = pl.ds(j * 16, 16)
                xjb[b3, e, sl] = xjb[b3, e, sl] * wcb[b2, e, sl]
            return carry2
        lax.fori_loop(0, CHUNK, mul, 0, unroll=4)

        pltpu.async_copy(xjb.at[b3], shared.at[rb2.at[b3]], ssem.at[b3],
                         add=True)

        for k in range(1, NCHUNK // IBLK):
            @pl.when(ch == k * IBLK - 2)
            def _():
                stage_iblk(k)

        nxt = ch + 2

        @pl.when(nxt < NCHUNK)
        def _():
            nb3 = lax.rem(nxt, 3)
            nb2 = lax.rem(nxt, 2)

            @pl.when(ch >= 1)
            def _():
                wait_s(nb3)
            loff = lax.rem(nxt, IBLK) * CHUNK
            cslot = jnp.where(lax.rem(nxt, 2 * IBLK) >= IBLK, 1, 0)
            build_rb2(nb3, loff)
            issue_gw(nxt, cslot, loff, nb3, nb2)
        return carry

    lax.fori_loop(0, NCHUNK, body, 0)
    for ch in range(NCHUNK - 3, NCHUNK):
        wait_s(ch % 3)
    plsc.subcore_barrier()
    pltpu.sync_copy(shared.at[pl.ds(r0, ROWS_PER_SUB)],
                    agg_hbm.at[c, pl.ds(r0, ROWS_PER_SUB)])


@jax.jit
def _sc_gather_scatter(x1, wc, row2d, col2d):
    mesh = plsc.VectorSubcoreMesh(core_axis_name="c", subcore_axis_name="s",
                                  num_cores=NC, num_subcores=NS)
    return pl.kernel(
        _sc_gs_body,
        out_type=jax.ShapeDtypeStruct((NC, NP, F), jnp.float32),
        mesh=mesh,
        compiler_params=pltpu.CompilerParams(needs_layout_passes=False),
        scratch_types=[
            pltpu.VMEM((2, IBLK * CHUNK), jnp.int32),
            pltpu.VMEM((IBLK * CHUNK,), jnp.int32),
            pltpu.VMEM((3, CHUNK), jnp.int32),
            pltpu.VMEM((3, CHUNK, F), jnp.float32),
            pltpu.VMEM((2, CHUNK, F), jnp.float32),
            pltpu.VMEM_SHARED((NP, F), jnp.float32),
            pltpu.SemaphoreType.DMA((3,)),
            pltpu.SemaphoreType.DMA((2,)),
            pltpu.SemaphoreType.DMA((3,)),
        ],
    )(x1, wc, row2d, col2d)


def _tc_edgeprep_kernel(d2_ref, d_ref, c_ref):
    i = pl.program_id(0)
    d = jnp.sqrt(d2_ref[...] + 1e-12)
    C = 0.5 * (jnp.cos(d * (np.pi / CUT)) + 1.0)
    C = jnp.where(d < CUT, C, 0.0)
    eidx = (i * 8192
            + lax.broadcasted_iota(jnp.int32, (64, 128), 0) * 128
            + lax.broadcasted_iota(jnp.int32, (64, 128), 1))
    C = jnp.where(eidx < E, C, 0.0)
    d_ref[...] = d
    c_ref[...] = C


@jax.jit
def _tc_edgeprep(d2r):
    return pl.pallas_call(
        _tc_edgeprep_kernel,
        grid=(EP // 8192,),
        in_specs=[pl.BlockSpec((64, 128), lambda i: (i, 0))],
        out_specs=[pl.BlockSpec((64, 128), lambda i: (i, 0)),
                   pl.BlockSpec((64, 128), lambda i: (i, 0))],
        out_shape=[jax.ShapeDtypeStruct((EP // 128, 128), jnp.float32),
                   jax.ShapeDtypeStruct((EP // 128, 128), jnp.float32)],
    )(d2r)


def _tc_filter_kernel(d_ref, c_ref, w1_ref, b1_ref, w2_ref, b2_ref, wc_ref):
    d = d_ref[...]
    j = lax.broadcasted_iota(jnp.int32, (NB, 64), 1).astype(jnp.float32)
    rbf = jnp.exp(_COEFF * (d - j * _STEP) ** 2)
    t = _ssp(jnp.dot(rbf, w1_ref[...], preferred_element_type=jnp.float32)
             + b1_ref[...])
    W = jnp.dot(t, w2_ref[...], preferred_element_type=jnp.float32) + b2_ref[...]
    wc_ref[...] = W * c_ref[...]


@jax.jit
def _tc_filter(d_c, c_c, w1p, b1, w2, b2):
    return pl.pallas_call(
        _tc_filter_kernel,
        grid=(EP // NB,),
        in_specs=[
            pl.BlockSpec((NB, 1), lambda i: (i, 0)),
            pl.BlockSpec((NB, 1), lambda i: (i, 0)),
            pl.BlockSpec((64, F), lambda i: (0, 0)),
            pl.BlockSpec((1, F), lambda i: (0, 0)),
            pl.BlockSpec((F, F), lambda i: (0, 0)),
            pl.BlockSpec((1, F), lambda i: (0, 0)),
        ],
        out_specs=pl.BlockSpec((NB, F), lambda i: (i, 0)),
        out_shape=jax.ShapeDtypeStruct((EP, F), jnp.float32),
    )(d_c, c_c, w1p, b1, w2, b2)


def _tc_prologue_kernel(az_ref, emb_ref, l1_ref, h_ref, x1_ref):
    az = az_ref[...]
    j = lax.broadcasted_iota(jnp.int32, (NB, 128), 1).astype(jnp.float32)
    oh = (az == j).astype(jnp.float32)
    h = jnp.dot(oh, emb_ref[...], preferred_element_type=jnp.float32)
    h_ref[...] = h
    x1_ref[...] = jnp.dot(h, l1_ref[...], preferred_element_type=jnp.float32)


@jax.jit
def _tc_prologue(azf, embp, l1):
    return pl.pallas_call(
        _tc_prologue_kernel,
        grid=(NP // NB,),
        in_specs=[
            pl.BlockSpec((NB, 1), lambda i: (i, 0)),
            pl.BlockSpec((128, H), lambda i: (0, 0)),
            pl.BlockSpec((H, F), lambda i: (0, 0)),
        ],
        out_specs=[pl.BlockSpec((NB, H), lambda i: (i, 0)),
                   pl.BlockSpec((NB, F), lambda i: (i, 0))],
        out_shape=[jax.ShapeDtypeStruct((NP, H), jnp.float32),
                   jax.ShapeDtypeStruct((NP, F), jnp.float32)],
    )(azf, embp, l1)


def _tc_node_kernel(a_ref, h_ref, w2_ref, b2_ref, w3_ref, b3_ref, l1_ref,
                    hn_ref, x1_ref):
    i = pl.program_id(0)
    agg = a_ref[0] + a_ref[1]
    x = _ssp(jnp.dot(agg, w2_ref[...], preferred_element_type=jnp.float32)
             + b2_ref[...])
    x = jnp.dot(x, w3_ref[...], preferred_element_type=jnp.float32) + b3_ref[...]
    ridx = i * NB + lax.broadcasted_iota(jnp.int32, (NB, 1), 0)
    x = jnp.where(ridx < N, x, 0.0)
    hn = h_ref[...] + x
    hn_ref[...] = hn
    x1_ref[...] = jnp.dot(hn, l1_ref[...], preferred_element_type=jnp.float32)


@jax.jit
def _tc_node_update(aggp, h, w2, b2, w3, b3, l1n):
    return pl.pallas_call(
        _tc_node_kernel,
        grid=(NP // NB,),
        in_specs=[
            pl.BlockSpec((2, NB, F), lambda i: (0, i, 0)),
            pl.BlockSpec((NB, H), lambda i: (i, 0)),
            pl.BlockSpec((F, H), lambda i: (0, 0)),
            pl.BlockSpec((1, H), lambda i: (0, 0)),
            pl.BlockSpec((H, H), lambda i: (0, 0)),
            pl.BlockSpec((1, H), lambda i: (0, 0)),
            pl.BlockSpec((H, F), lambda i: (0, 0)),
        ],
        out_specs=[pl.BlockSpec((NB, H), lambda i: (i, 0)),
                   pl.BlockSpec((NB, F), lambda i: (i, 0))],
        out_shape=[jax.ShapeDtypeStruct((NP, H), jnp.float32),
                   jax.ShapeDtypeStruct((NP, F), jnp.float32)],
    )(aggp, h, w2, b2, w3, b3, l1n)


def _tc_film_kernel(did_ref, de_ref, w1_ref, b1_ref, w2_ref, b2_ref,
                    gw_ref, gb_ref, bw_ref, bb_ref, gam_ref, bet_ref):
    did = did_ref[...]
    j = lax.broadcasted_iota(jnp.int32, (G, 8), 1).astype(jnp.float32)
    oh = (did == j).astype(jnp.float32)
    de = jnp.dot(oh, de_ref[...], preferred_element_type=jnp.float32)
    fc = jax.nn.relu(jnp.dot(de, w1_ref[...], preferred_element_type=jnp.float32)
                     + b1_ref[...])
    fc = jnp.dot(fc, w2_ref[...], preferred_element_type=jnp.float32) + b2_ref[...]
    gam_ref[...] = jnp.dot(fc, gw_ref[...],
                           preferred_element_type=jnp.float32) + gb_ref[...]
    bet_ref[...] = jnp.dot(fc, bw_ref[...],
                           preferred_element_type=jnp.float32) + bb_ref[...]


@jax.jit
def _tc_film(didf, dep, fp1_w, fp1_b, fp2_w, fp2_b, gam_w, gam_b, bet_w, bet_b):
    full = lambda *s: pl.BlockSpec(s, lambda: tuple(0 for _ in s))
    return pl.pallas_call(
        _tc_film_kernel,
        in_specs=[full(G, 1), full(8, 64), full(64, 128), full(1, 128),
                  full(128, 128), full(1, 128), full(128, H), full(1, H),
                  full(128, H), full(1, H)],
        out_specs=[full(G, H), full(G, H)],
        out_shape=[jax.ShapeDtypeStruct((G, H), jnp.float32),
                   jax.ShapeDtypeStruct((G, H), jnp.float32)],
    )(didf, dep, fp1_w, fp1_b, fp2_w, fp2_b, gam_w, gam_b, bet_w, bet_b)


def _tc_final_kernel(h_ref, b_ref, gam_ref, bet_ref, w1_ref, b1_ref,
                     w2_ref, b2_ref, out_ref):
    i = pl.program_id(0)
    bf = b_ref[...]
    j = lax.broadcasted_iota(jnp.int32, (NB, G), 1).astype(jnp.float32)
    oh = (bf == j).astype(jnp.float32)
    gam_n = jnp.dot(oh, gam_ref[...], preferred_element_type=jnp.float32)
    bet_n = jnp.dot(oh, bet_ref[...], preferred_element_type=jnp.float32)
    hp = gam_n * h_ref[...] + bet_n
    t = _ssp(jnp.dot(hp, w1_ref[...], preferred_element_type=jnp.float32)
             + b1_ref[...])
    o = jnp.dot(t, w2_ref[...], preferred_element_type=jnp.float32) + b2_ref[...]
    part = lax.dot_general(oh, o, (((0,), (0,)), ((), ())),
                           preferred_element_type=jnp.float32)

    @pl.when(i == 0)
    def _():
        out_ref[...] = jnp.zeros_like(out_ref)

    out_ref[...] += part


@jax.jit
def _tc_final(h, batchf, gamma, beta, out1_w, out1_b, out2_wp, out2_bp):
    return pl.pallas_call(
        _tc_final_kernel,
        grid=(NP // NB,),
        in_specs=[
            pl.BlockSpec((NB, H), lambda i: (i, 0)),
            pl.BlockSpec((NB, 1), lambda i: (i, 0)),
            pl.BlockSpec((G, H), lambda i: (0, 0)),
            pl.BlockSpec((G, H), lambda i: (0, 0)),
            pl.BlockSpec((H, H // 2), lambda i: (0, 0)),
            pl.BlockSpec((1, H // 2), lambda i: (0, 0)),
            pl.BlockSpec((H // 2, 8), lambda i: (0, 0)),
            pl.BlockSpec((1, 8), lambda i: (0, 0)),
        ],
        out_specs=pl.BlockSpec((G, 8), lambda i: (0, 0)),
        out_shape=jax.ShapeDtypeStruct((G, 8), jnp.float32),
    )(h, batchf, gamma, beta, out1_w, out1_b, out2_wp, out2_bp)


def kernel(pos, atomic_numbers, batch, edge_index, domain_ids, emb, mlp_w1,
           mlp_b1, mlp_w2, mlp_b2, lin1_w, lin2_w, lin2_b, lin3_w, lin3_b,
           out1_w, out1_b, out2_w, out2_b, dom_emb, fp1_w, fp1_b, fp2_w,
           fp2_b, gam_w, gam_b, bet_w, bet_b):
    npad = EP - E
    pad_idx = (jnp.arange(npad, dtype=jnp.int32) * 7) % N
    row_p = jnp.concatenate([edge_index[0].astype(jnp.int32), pad_idx])
    col_p = jnp.concatenate([edge_index[1].astype(jnp.int32), pad_idx])
    row2d = row_p.reshape(NW, EPW)
    col2d = col_p.reshape(NW, EPW)
    pos_flat = pos.reshape(-1)

    azf = jnp.pad(atomic_numbers.astype(jnp.float32), (0, NP - N),
                  constant_values=float(ZMAX)).reshape(NP, 1)
    batchf = jnp.pad(batch.astype(jnp.float32), (0, NP - N),
                     constant_values=float(G)).reshape(NP, 1)
    didf = domain_ids.astype(jnp.float32).reshape(G, 1)

    w1p = jnp.pad(mlp_w1, ((0, 0), (0, 64 - NG), (0, 0)))
    embp = jnp.pad(emb, ((0, 128 - ZMAX), (0, 0)))
    dep = jnp.pad(dom_emb, ((0, 3), (0, 0)))
    out2_wp = jnp.pad(out2_w, ((0, 0), (0, 7)))
    out2_bp = jnp.pad(out2_b, (0, 7)).reshape(1, 8)
    r2 = lambda v: v.reshape(1, -1)

    d2 = _sc_dist(pos_flat, row_p, col_p)
    dd, cc = _tc_edgeprep(d2.reshape(EP // 128, 128))
    d_c = dd.reshape(EP, 1)
    c_c = cc.reshape(EP, 1)
    h, x1 = _tc_prologue(azf, embp, lin1_w[0])
    wcs = [_tc_filter(d_c, c_c, w1p[i], r2(mlp_b1[i]), mlp_w2[i],
                      r2(mlp_b2[i])) for i in range(NI)]
    for i in range(NI):
        aggp = _sc_gather_scatter(x1, wcs[i], row2d, col2d)
        h, x1 = _tc_node_update(aggp, h, lin2_w[i], r2(lin2_b[i]),
                                lin3_w[i], r2(lin3_b[i]),
                                lin1_w[(i + 1) % NI])
    gamma, beta = _tc_film(didf, dep, fp1_w, r2(fp1_b), fp2_w, r2(fp2_b),
                           gam_w, r2(gam_b), bet_w, r2(bet_b))
    eng = _tc_final(h, batchf, gamma, beta, out1_w, r2(out1_b),
                    out2_wp, out2_bp)
    return eng[:, :1]


# speedup vs baseline: 2.7782x; 1.0013x over previous
import functools

import jax
import jax.numpy as jnp
import numpy as np
from jax import lax
from jax.experimental import pallas as pl
from jax.experimental.pallas import tpu as pltpu
from jax.experimental.pallas import tpu_sc as plsc

N = 10000
E = 320000
H = 128
F = 128
NG = 50
NI = 6
G = 128
CUT = 10.0
ZMAX = 100

NC = 2
NS = 16
NW = NC * NS
CHUNK = 64
NCHUNK = 160
IBLK = 32
EPW = NCHUNK * CHUNK
EP = NW * EPW
NP = 10240
NB = 512
ROWS_PER_SUB = NP // NS

_LOG2 = float(np.log(2.0))
_STEP = CUT / (NG - 1)
_COEFF = -0.5 / _STEP ** 2


def _ssp(x):
    return jnp.where(x > 30.0, x - _LOG2,
                     _LOG2 * jnp.log2(0.5 + 0.5 * jnp.exp(x)))


def _sc_dist_body(pos_hbm, row_hbm, col_hbm, d2_hbm, pos_t, rowb, colb, d2b):
    wid = lax.axis_index("s") * NC + lax.axis_index("c")
    base = wid * EPW
    pltpu.sync_copy(pos_hbm, pos_t)
    pltpu.sync_copy(row_hbm.at[pl.ds(base, EPW)], rowb)
    pltpu.sync_copy(col_hbm.at[pl.ds(base, EPW)], colb)

    def body(g, carry):
        sl = pl.ds(g * 16, 16)
        r3 = rowb[sl] * 3
        c3 = colb[sl] * 3
        dx = plsc.load_gather(pos_t, [r3]) - plsc.load_gather(pos_t, [c3])
        dy = plsc.load_gather(pos_t, [r3 + 1]) - plsc.load_gather(pos_t, [c3 + 1])
        dz = plsc.load_gather(pos_t, [r3 + 2]) - plsc.load_gather(pos_t, [c3 + 2])
        d2b[sl] = dx * dx + dy * dy + dz * dz
        return carry

    lax.fori_loop(0, EPW // 16, body, 0, unroll=2)
    pltpu.sync_copy(d2b, d2_hbm.at[pl.ds(base, EPW)])


@jax.jit
def _sc_dist(pos_flat, row_p, col_p):
    mesh = plsc.VectorSubcoreMesh(core_axis_name="c", subcore_axis_name="s",
                                  num_cores=NC, num_subcores=NS)
    return pl.kernel(
        _sc_dist_body,
        out_type=jax.ShapeDtypeStruct((EP,), jnp.float32),
        mesh=mesh,
        compiler_params=pltpu.CompilerParams(needs_layout_passes=False),
        scratch_types=[
            pltpu.VMEM((N * 3,), jnp.float32),
            pltpu.VMEM((EPW,), jnp.int32),
            pltpu.VMEM((EPW,), jnp.int32),
            pltpu.VMEM((EPW,), jnp.float32),
        ],
    )(pos_flat, row_p, col_p)


def _sc_gs_body(x1_hbm, wc_hbm, row_hbm, col_hbm, agg_hbm,
                colb, rowb, rb2, xjb, wcb, shared, gsem, wsem, ssem):
    c = lax.axis_index("c")
    s = lax.axis_index("s")
    wid = s * NC + c
    ebase = wid * EPW
    r0 = s * ROWS_PER_SUB

    def stage_iblk(k):
        off = k * IBLK * CHUNK
        pltpu.sync_copy(col_hbm.at[wid, pl.ds(off, IBLK * CHUNK)],
                        colb.at[k % 2])
        pltpu.sync_copy(row_hbm.at[wid, pl.ds(off, IBLK * CHUNK)], rowb)

    def build_rb2(slot, loff):
        def ib(g, carry2):
            rb2[slot, pl.ds(g * 16, 16)] = rowb[pl.ds(loff + g * 16, 16)]
            return carry2
        lax.fori_loop(0, CHUNK // 16, ib, 0)

    def issue_gw(ch, cslot, loff, slot3, slot2):
        g = pltpu.async_copy(
            x1_hbm.at[colb.at[cslot, pl.ds(loff, CHUNK)]], xjb.at[slot3],
            gsem.at[slot3])
        w = pltpu.async_copy(
            wc_hbm.at[pl.ds(ebase + ch * CHUNK, CHUNK)], wcb.at[slot2],
            wsem.at[slot2])
        return g, w

    def wait_g(slot3):
        pltpu.make_async_copy(x1_hbm.at[pl.ds(0, CHUNK)], xjb.at[slot3],
                              gsem.at[slot3]).wait()

    def wait_w(slot2):
        pltpu.make_async_copy(wc_hbm.at[pl.ds(0, CHUNK)], wcb.at[slot2],
                              wsem.at[slot2]).wait()

    def wait_s(slot3):
        pltpu.make_async_copy(wc_hbm.at[pl.ds(0, CHUNK)], xjb.at[slot3],
                              ssem.at[slot3]).wait()

    def zbody(e, carry):
        for j in range(8):
            xjb[0, e, pl.ds(j * 16, 16)] = jnp.zeros((16,), jnp.float32)
        return carry
    lax.fori_loop(0, CHUNK, zbody, 0, unroll=2)
    for k in range(ROWS_PER_SUB // CHUNK):
        pltpu.sync_copy(xjb.at[0], shared.at[pl.ds(r0 + k * CHUNK, CHUNK)])
    plsc.subcore_barrier()

    stage_iblk(0)
    for ch in range(2):
        build_rb2(ch, ch * CHUNK)
        issue_gw(ch, 0, ch * CHUNK, ch, ch)

    def body(ch, carry):
        b3 = lax.rem(ch, 3)
        b2 = lax.rem(ch, 2)
        wait_g(b3)
        wait_w(b2)

        def mul(e, carry2):
            for j in range(8):
                sl = pl.ds(j * 16, 16)
                xjb[b3, e, sl] = xjb[b3, e, sl] * wcb[b2, e, sl]
            return carry2
        lax.fori_loop(0, CHUNK, mul, 0, unroll=4)

        pltpu.async_copy(xjb.at[b3], shared.at[rb2.at[b3]], ssem.at[b3],
                         add=True)

        for k in range(1, NCHUNK // IBLK):
            @pl.when(ch == k * IBLK - 2)
            def _():
                stage_iblk(k)

        nxt = ch + 2

        @pl.when(nxt < NCHUNK)
        def _():
            nb3 = lax.rem(nxt, 3)
            nb2 = lax.rem(nxt, 2)

            @pl.when(ch >= 1)
            def _():
                wait_s(nb3)
            loff = lax.rem(nxt, IBLK) * CHUNK
            cslot = jnp.where(lax.rem(nxt, 2 * IBLK) >= IBLK, 1, 0)
            build_rb2(nb3, loff)
            issue_gw(nxt, cslot, loff, nb3, nb2)
        return carry

    lax.fori_loop(0, NCHUNK, body, 0)
    for ch in range(NCHUNK - 3, NCHUNK):
        wait_s(ch % 3)
    plsc.subcore_barrier()
    pltpu.sync_copy(shared.at[pl.ds(r0, ROWS_PER_SUB)],
                    agg_hbm.at[c, pl.ds(r0, ROWS_PER_SUB)])


@jax.jit
def _sc_gather_scatter(x1, wc, row2d, col2d):
    mesh = plsc.VectorSubcoreMesh(core_axis_name="c", subcore_axis_name="s",
                                  num_cores=NC, num_subcores=NS)
    return pl.kernel(
        _sc_gs_body,
        out_type=jax.ShapeDtypeStruct((NC, NP, F), jnp.float32),
        mesh=mesh,
        compiler_params=pltpu.CompilerParams(needs_layout_passes=False,
                                             use_tc_tiling_on_sc=True),
        scratch_types=[
            pltpu.VMEM((2, IBLK * CHUNK), jnp.int32),
            pltpu.VMEM((IBLK * CHUNK,), jnp.int32),
            pltpu.VMEM((3, CHUNK), jnp.int32),
            pltpu.VMEM((3, CHUNK, F), jnp.float32),
            pltpu.VMEM((2, CHUNK, F), jnp.float32),
            pltpu.VMEM_SHARED((NP, F), jnp.float32),
            pltpu.SemaphoreType.DMA((3,)),
            pltpu.SemaphoreType.DMA((2,)),
            pltpu.SemaphoreType.DMA((3,)),
        ],
    )(x1, wc, row2d, col2d)


def _tc_edgeprep_kernel(d2_ref, d_ref, c_ref):
    i = pl.program_id(0)
    d = jnp.sqrt(d2_ref[...] + 1e-12)
    C = 0.5 * (jnp.cos(d * (np.pi / CUT)) + 1.0)
    C = jnp.where(d < CUT, C, 0.0)
    eidx = (i * 8192
            + lax.broadcasted_iota(jnp.int32, (64, 128), 0) * 128
            + lax.broadcasted_iota(jnp.int32, (64, 128), 1))
    C = jnp.where(eidx < E, C, 0.0)
    d_ref[...] = d
    c_ref[...] = C


@jax.jit
def _tc_edgeprep(d2r):
    return pl.pallas_call(
        _tc_edgeprep_kernel,
        grid=(EP // 8192,),
        in_specs=[pl.BlockSpec((64, 128), lambda i: (i, 0))],
        out_specs=[pl.BlockSpec((64, 128), lambda i: (i, 0)),
                   pl.BlockSpec((64, 128), lambda i: (i, 0))],
        out_shape=[jax.ShapeDtypeStruct((EP // 128, 128), jnp.float32),
                   jax.ShapeDtypeStruct((EP // 128, 128), jnp.float32)],
    )(d2r)


def _tc_filter_kernel(d_ref, c_ref, w1_ref, b1_ref, w2_ref, b2_ref, wc_ref):
    d = d_ref[...]
    j = lax.broadcasted_iota(jnp.int32, (NB, 64), 1).astype(jnp.float32)
    rbf = jnp.exp(_COEFF * (d - j * _STEP) ** 2)
    t = _ssp(jnp.dot(rbf, w1_ref[...], preferred_element_type=jnp.float32)
             + b1_ref[...])
    W = jnp.dot(t, w2_ref[...], preferred_element_type=jnp.float32) + b2_ref[...]
    wc_ref[...] = W * c_ref[...]


@jax.jit
def _tc_filter(d_c, c_c, w1p, b1, w2, b2):
    return pl.pallas_call(
        _tc_filter_kernel,
        grid=(EP // NB,),
        in_specs=[
            pl.BlockSpec((NB, 1), lambda i: (i, 0)),
            pl.BlockSpec((NB, 1), lambda i: (i, 0)),
            pl.BlockSpec((64, F), lambda i: (0, 0)),
            pl.BlockSpec((1, F), lambda i: (0, 0)),
            pl.BlockSpec((F, F), lambda i: (0, 0)),
            pl.BlockSpec((1, F), lambda i: (0, 0)),
        ],
        out_specs=pl.BlockSpec((NB, F), lambda i: (i, 0)),
        out_shape=jax.ShapeDtypeStruct((EP, F), jnp.float32),
    )(d_c, c_c, w1p, b1, w2, b2)


def _tc_prologue_kernel(az_ref, emb_ref, l1_ref, h_ref, x1_ref):
    az = az_ref[...]
    j = lax.broadcasted_iota(jnp.int32, (NB, 128), 1).astype(jnp.float32)
    oh = (az == j).astype(jnp.float32)
    h = jnp.dot(oh, emb_ref[...], preferred_element_type=jnp.float32)
    h_ref[...] = h
    x1_ref[...] = jnp.dot(h, l1_ref[...], preferred_element_type=jnp.float32)


@jax.jit
def _tc_prologue(azf, embp, l1):
    return pl.pallas_call(
        _tc_prologue_kernel,
        grid=(NP // NB,),
        in_specs=[
            pl.BlockSpec((NB, 1), lambda i: (i, 0)),
            pl.BlockSpec((128, H), lambda i: (0, 0)),
            pl.BlockSpec((H, F), lambda i: (0, 0)),
        ],
        out_specs=[pl.BlockSpec((NB, H), lambda i: (i, 0)),
                   pl.BlockSpec((NB, F), lambda i: (i, 0))],
        out_shape=[jax.ShapeDtypeStruct((NP, H), jnp.float32),
                   jax.ShapeDtypeStruct((NP, F), jnp.float32)],
    )(azf, embp, l1)


def _tc_node_kernel(a_ref, h_ref, w2_ref, b2_ref, w3_ref, b3_ref, l1_ref,
                    hn_ref, x1_ref):
    i = pl.program_id(0)
    agg = a_ref[0] + a_ref[1]
    x = _ssp(jnp.dot(agg, w2_ref[...], preferred_element_type=jnp.float32)
             + b2_ref[...])
    x = jnp.dot(x, w3_ref[...], preferred_element_type=jnp.float32) + b3_ref[...]
    ridx = i * NB + lax.broadcasted_iota(jnp.int32, (NB, 1), 0)
    x = jnp.where(ridx < N, x, 0.0)
    hn = h_ref[...] + x
    hn_ref[...] = hn
    x1_ref[...] = jnp.dot(hn, l1_ref[...], preferred_element_type=jnp.float32)


@jax.jit
def _tc_node_update(aggp, h, w2, b2, w3, b3, l1n):
    return pl.pallas_call(
        _tc_node_kernel,
        grid=(NP // NB,),
        in_specs=[
            pl.BlockSpec((2, NB, F), lambda i: (0, i, 0)),
            pl.BlockSpec((NB, H), lambda i: (i, 0)),
            pl.BlockSpec((F, H), lambda i: (0, 0)),
            pl.BlockSpec((1, H), lambda i: (0, 0)),
            pl.BlockSpec((H, H), lambda i: (0, 0)),
            pl.BlockSpec((1, H), lambda i: (0, 0)),
            pl.BlockSpec((H, F), lambda i: (0, 0)),
        ],
        out_specs=[pl.BlockSpec((NB, H), lambda i: (i, 0)),
                   pl.BlockSpec((NB, F), lambda i: (i, 0))],
        out_shape=[jax.ShapeDtypeStruct((NP, H), jnp.float32),
                   jax.ShapeDtypeStruct((NP, F), jnp.float32)],
    )(aggp, h, w2, b2, w3, b3, l1n)


def _tc_film_kernel(did_ref, de_ref, w1_ref, b1_ref, w2_ref, b2_ref,
                    gw_ref, gb_ref, bw_ref, bb_ref, gam_ref, bet_ref):
    did = did_ref[...]
    j = lax.broadcasted_iota(jnp.int32, (G, 8), 1).astype(jnp.float32)
    oh = (did == j).astype(jnp.float32)
    de = jnp.dot(oh, de_ref[...], preferred_element_type=jnp.float32)
    fc = jax.nn.relu(jnp.dot(de, w1_ref[...], preferred_element_type=jnp.float32)
                     + b1_ref[...])
    fc = jnp.dot(fc, w2_ref[...], preferred_element_type=jnp.float32) + b2_ref[...]
    gam_ref[...] = jnp.dot(fc, gw_ref[...],
                           preferred_element_type=jnp.float32) + gb_ref[...]
    bet_ref[...] = jnp.dot(fc, bw_ref[...],
                           preferred_element_type=jnp.float32) + bb_ref[...]


@jax.jit
def _tc_film(didf, dep, fp1_w, fp1_b, fp2_w, fp2_b, gam_w, gam_b, bet_w, bet_b):
    full = lambda *s: pl.BlockSpec(s, lambda: tuple(0 for _ in s))
    return pl.pallas_call(
        _tc_film_kernel,
        in_specs=[full(G, 1), full(8, 64), full(64, 128), full(1, 128),
                  full(128, 128), full(1, 128), full(128, H), full(1, H),
                  full(128, H), full(1, H)],
        out_specs=[full(G, H), full(G, H)],
        out_shape=[jax.ShapeDtypeStruct((G, H), jnp.float32),
                   jax.ShapeDtypeStruct((G, H), jnp.float32)],
    )(didf, dep, fp1_w, fp1_b, fp2_w, fp2_b, gam_w, gam_b, bet_w, bet_b)


def _tc_final_kernel(h_ref, b_ref, gam_ref, bet_ref, w1_ref, b1_ref,
                     w2_ref, b2_ref, out_ref):
    i = pl.program_id(0)
    bf = b_ref[...]
    j = lax.broadcasted_iota(jnp.int32, (NB, G), 1).astype(jnp.float32)
    oh = (bf == j).astype(jnp.float32)
    gam_n = jnp.dot(oh, gam_ref[...], preferred_element_type=jnp.float32)
    bet_n = jnp.dot(oh, bet_ref[...], preferred_element_type=jnp.float32)
    hp = gam_n * h_ref[...] + bet_n
    t = _ssp(jnp.dot(hp, w1_ref[...], preferred_element_type=jnp.float32)
             + b1_ref[...])
    o = jnp.dot(t, w2_ref[...], preferred_element_type=jnp.float32) + b2_ref[...]
    part = lax.dot_general(oh, o, (((0,), (0,)), ((), ())),
                           preferred_element_type=jnp.float32)

    @pl.when(i == 0)
    def _():
        out_ref[...] = jnp.zeros_like(out_ref)

    out_ref[...] += part


@jax.jit
def _tc_final(h, batchf, gamma, beta, out1_w, out1_b, out2_wp, out2_bp):
    return pl.pallas_call(
        _tc_final_kernel,
        grid=(NP // NB,),
        in_specs=[
            pl.BlockSpec((NB, H), lambda i: (i, 0)),
            pl.BlockSpec((NB, 1), lambda i: (i, 0)),
            pl.BlockSpec((G, H), lambda i: (0, 0)),
            pl.BlockSpec((G, H), lambda i: (0, 0)),
            pl.BlockSpec((H, H // 2), lambda i: (0, 0)),
            pl.BlockSpec((1, H // 2), lambda i: (0, 0)),
            pl.BlockSpec((H // 2, 8), lambda i: (0, 0)),
            pl.BlockSpec((1, 8), lambda i: (0, 0)),
        ],
        out_specs=pl.BlockSpec((G, 8), lambda i: (0, 0)),
        out_shape=jax.ShapeDtypeStruct((G, 8), jnp.float32),
    )(h, batchf, gamma, beta, out1_w, out1_b, out2_wp, out2_bp)


def kernel(pos, atomic_numbers, batch, edge_index, domain_ids, emb, mlp_w1,
           mlp_b1, mlp_w2, mlp_b2, lin1_w, lin2_w, lin2_b, lin3_w, lin3_b,
           out1_w, out1_b, out2_w, out2_b, dom_emb, fp1_w, fp1_b, fp2_w,
           fp2_b, gam_w, gam_b, bet_w, bet_b):
    npad = EP - E
    pad_idx = (jnp.arange(npad, dtype=jnp.int32) * 7) % N
    row_p = jnp.concatenate([edge_index[0].astype(jnp.int32), pad_idx])
    col_p = jnp.concatenate([edge_index[1].astype(jnp.int32), pad_idx])
    row2d = row_p.reshape(NW, EPW)
    col2d = col_p.reshape(NW, EPW)
    pos_flat = pos.reshape(-1)

    azf = jnp.pad(atomic_numbers.astype(jnp.float32), (0, NP - N),
                  constant_values=float(ZMAX)).reshape(NP, 1)
    batchf = jnp.pad(batch.astype(jnp.float32), (0, NP - N),
                     constant_values=float(G)).reshape(NP, 1)
    didf = domain_ids.astype(jnp.float32).reshape(G, 1)

    w1p = jnp.pad(mlp_w1, ((0, 0), (0, 64 - NG), (0, 0)))
    embp = jnp.pad(emb, ((0, 128 - ZMAX), (0, 0)))
    dep = jnp.pad(dom_emb, ((0, 3), (0, 0)))
    out2_wp = jnp.pad(out2_w, ((0, 0), (0, 7)))
    out2_bp = jnp.pad(out2_b, (0, 7)).reshape(1, 8)
    r2 = lambda v: v.reshape(1, -1)

    d2 = _sc_dist(pos_flat, row_p, col_p)
    dd, cc = _tc_edgeprep(d2.reshape(EP // 128, 128))
    d_c = dd.reshape(EP, 1)
    c_c = cc.reshape(EP, 1)
    h, x1 = _tc_prologue(azf, embp, lin1_w[0])
    wcs = [_tc_filter(d_c, c_c, w1p[i], r2(mlp_b1[i]), mlp_w2[i],
                      r2(mlp_b2[i])) for i in range(NI)]
    for i in range(NI):
        aggp = _sc_gather_scatter(x1, wcs[i], row2d, col2d)
        h, x1 = _tc_node_update(aggp, h, lin2_w[i], r2(lin2_b[i]),
                                lin3_w[i], r2(lin3_b[i]),
                                lin1_w[(i + 1) % NI])
    gamma, beta = _tc_film(didf, dep, fp1_w, r2(fp1_b), fp2_w, r2(fp2_b),
                           gam_w, r2(gam_b), bet_w, r2(bet_b))
    eng = _tc_final(h, batchf, gamma, beta, out1_w, r2(out1_b),
                    out2_wp, out2_bp)
    return eng[:, :1]
